# Initial kernel scaffold; baseline (speedup 1.0000x reference)
import jax
import jax.numpy as jnp
from jax.experimental import pallas as pl


def kernel(x, edge_index, Wl0, Wr0, b0, Wl1, Wr1, b1, Wl2, Wr2, b2, Wm1, bm1, Wm2, bm2):
    raise NotImplementedError("write your pallas kernel here")


# speedup vs baseline: 7.9203x; 7.9203x over previous
import functools

import jax
import jax.numpy as jnp
from jax import lax
from jax.experimental import pallas as pl
from jax.experimental.pallas import tpu as pltpu
from jax.experimental.pallas import tpu_sc as plsc

N = 10000
E = 320000
D = 128
B = 100
MLP_H = 256
OUT = 64

NC = 2
NS = 16
NW = NC * NS
EPW = E // NW
CH = 125
NCHUNK = EPW // CH
NP_ = 10240
RPW = NP_ // NS
ZCH = 64


def _sc_agg(h, srcr, dstr, with_cnt):
    mesh = plsc.VectorSubcoreMesh(core_axis_name="c", subcore_axis_name="s")
    out_type = [jax.ShapeDtypeStruct((NC, NP_, D), jnp.float32)]
    if with_cnt:
        out_type.append(jax.ShapeDtypeStruct((NC, NP_), jnp.float32))
    scratch = [
        pltpu.VMEM((NCHUNK, CH), jnp.int32),
        pltpu.VMEM((NCHUNK, CH), jnp.int32),
        pltpu.VMEM((CH, D), jnp.float32),
        pltpu.VMEM((ZCH, D), jnp.float32),
        pltpu.SemaphoreType.DMA,
        pltpu.VMEM_SHARED((NP_, D), jnp.float32),
    ]
    if with_cnt:
        scratch += [
            pltpu.VMEM((128,), jnp.float32),
            pltpu.VMEM((RPW,), jnp.float32),
            pltpu.VMEM_SHARED((NP_,), jnp.float32),
        ]

    @functools.partial(pl.kernel, mesh=mesh, out_type=tuple(out_type),
                       scratch_types=scratch)
    def k(h_hbm, src_hbm, dst_hbm, *refs):
        if with_cnt:
            (agg_out, cnt_out, src_v, dst_v, rows_v, zer_v, sem, acc,
             ones_v, z16_v, cacc) = refs
        else:
            (agg_out, src_v, dst_v, rows_v, zer_v, sem, acc) = refs
        c = lax.axis_index("c")
        s = lax.axis_index("s")
        wid = c * NS + s

        zero16 = jnp.zeros((16,), jnp.float32)

        def zrow(i, carry):
            for jj in range(D // 16):
                zer_v[i, jj * 16:(jj + 1) * 16] = zero16
            return carry
        lax.fori_loop(0, ZCH, zrow, 0)
        for t in range(RPW // ZCH):
            pltpu.sync_copy(zer_v, acc.at[pl.ds(s * RPW + t * ZCH, ZCH)])
        if with_cnt:
            one16 = jnp.ones((16,), jnp.float32)
            for jj in range(RPW // 16):
                z16_v[jj * 16:(jj + 1) * 16] = zero16
            for jj in range(128 // 16):
                ones_v[jj * 16:(jj + 1) * 16] = one16
            pltpu.sync_copy(z16_v, cacc.at[pl.ds(s * RPW, RPW)])
        plsc.subcore_barrier()

        pltpu.sync_copy(src_hbm.at[wid], src_v)
        pltpu.sync_copy(dst_hbm.at[wid], dst_v)

        def edge_chunk(j, carry):
            pltpu.async_copy(h_hbm.at[src_v.at[j]], rows_v, sem).wait()
            pltpu.sync_copy(rows_v, acc.at[dst_v.at[j]], add=True)
            if with_cnt:
                pltpu.sync_copy(ones_v.at[pl.ds(0, CH)],
                                cacc.at[dst_v.at[j]], add=True)
            return carry
        lax.fori_loop(0, NCHUNK, edge_chunk, 0)

        plsc.subcore_barrier()
        pltpu.sync_copy(acc.at[pl.ds(s * RPW, RPW)],
                        agg_out.at[c, pl.ds(s * RPW, RPW)])
        if with_cnt:
            pltpu.sync_copy(cacc.at[pl.ds(s * RPW, RPW)],
                            cnt_out.at[c, pl.ds(s * RPW, RPW)])

    res = k(h, srcr, dstr)
    return res if with_cnt else res[0]


def _tc_stage(agg, cnt, h, Wl, Wr, b):
    BR = 1000

    def body(a_ref, c_ref, h_ref, wl_ref, wr_ref, b_ref, o_ref):
        a = a_ref[0] + a_ref[1]
        deg = c_ref[0] + c_ref[1]
        mean = a * (1.0 / jnp.maximum(deg, 1.0))
        o_ref[...] = (
            jnp.dot(mean, wl_ref[...], preferred_element_type=jnp.float32)
            + jnp.dot(h_ref[...], wr_ref[...], preferred_element_type=jnp.float32)
            + b_ref[...])

    return pl.pallas_call(
        body,
        grid=(N // BR,),
        in_specs=[
            pl.BlockSpec((NC, BR, D), lambda i: (0, i, 0)),
            pl.BlockSpec((NC, BR, 1), lambda i: (0, i, 0)),
            pl.BlockSpec((BR, D), lambda i: (i, 0)),
            pl.BlockSpec((D, D), lambda i: (0, 0)),
            pl.BlockSpec((D, D), lambda i: (0, 0)),
            pl.BlockSpec((1, D), lambda i: (0, 0)),
        ],
        out_specs=pl.BlockSpec((BR, D), lambda i: (i, 0)),
        out_shape=jax.ShapeDtypeStruct((N, D), jnp.float32),
    )(agg, cnt.reshape(NC, NP_, 1), h, Wl, Wr, b.reshape(1, D))


def _mlp_head(hr, Wm1, bm1, Wm2, bm2):
    def body(h_ref, w1_ref, b1_ref, w2_ref, b2_ref, o_ref):
        z = jnp.maximum(
            jnp.dot(h_ref[...], w1_ref[...], preferred_element_type=jnp.float32)
            + b1_ref[...], 0.0)
        o_ref[...] = (jnp.dot(z, w2_ref[...], preferred_element_type=jnp.float32)
                      + b2_ref[...])

    return pl.pallas_call(
        body,
        out_shape=jax.ShapeDtypeStruct((B, OUT), jnp.float32),
    )(hr, Wm1, bm1.reshape(1, MLP_H), Wm2, bm2.reshape(1, OUT))


def kernel(x, edge_index, Wl0, Wr0, b0, Wl1, Wr1, b1, Wl2, Wr2, b2,
           Wm1, bm1, Wm2, bm2):
    ei = edge_index.astype(jnp.int32)
    srcr = ei[0].reshape(NW, NCHUNK, CH)
    dstr = ei[1].reshape(NW, NCHUNK, CH)

    agg1, cnt = _sc_agg(x, srcr, dstr, with_cnt=True)
    h1 = _tc_stage(agg1, cnt, x, Wl0, Wr0, b0)
    agg2 = _sc_agg(h1, srcr, dstr, with_cnt=False)
    h2 = _tc_stage(agg2, cnt, h1, Wl1, Wr1, b1)
    agg3 = _sc_agg(h2, srcr, dstr, with_cnt=False)
    h3 = _tc_stage(agg3, cnt, h2, Wl2, Wr2, b2)

    h3r = h3.reshape(B, N // B, D)[:, 0, :]
    return _mlp_head(h3r, Wm1, bm1, Wm2, bm2)


# speedup vs baseline: 11.9729x; 1.5117x over previous
import functools

import jax
import jax.numpy as jnp
from jax import lax
from jax.experimental import pallas as pl
from jax.experimental.pallas import tpu as pltpu
from jax.experimental.pallas import tpu_sc as plsc

N = 10000
E = 320000
D = 128
B = 100
MLP_H = 256
OUT = 64

NC = 2
NS = 16
NW = NC * NS
EPW = E // NW
CH = 125
NCHUNK = EPW // CH
NP_ = 10240
RPW = NP_ // NS
ZCH = 64


def _sc_agg(h, srcr, dstr, with_cnt):
    mesh = plsc.VectorSubcoreMesh(core_axis_name="c", subcore_axis_name="s")
    out_type = [jax.ShapeDtypeStruct((NC, NP_, D), jnp.float32)]
    if with_cnt:
        out_type.append(jax.ShapeDtypeStruct((NC, NP_), jnp.float32))
    scratch = [
        pltpu.VMEM((2, 8, CH), jnp.int32),
        pltpu.VMEM((2, 8, CH), jnp.int32),
        pltpu.VMEM((2, CH, D), jnp.float32),
        pltpu.VMEM((ZCH, D), jnp.float32),
        pltpu.SemaphoreType.DMA,
        pltpu.SemaphoreType.DMA,
        pltpu.SemaphoreType.DMA,
        pltpu.SemaphoreType.DMA,
        pltpu.SemaphoreType.DMA,
        pltpu.VMEM_SHARED((NP_, D), jnp.float32),
    ]
    if with_cnt:
        scratch += [
            pltpu.SemaphoreType.DMA,
            pltpu.SemaphoreType.DMA,
            pltpu.VMEM((128,), jnp.float32),
            pltpu.VMEM((RPW,), jnp.float32),
            pltpu.VMEM_SHARED((NP_,), jnp.float32),
        ]

    @functools.partial(pl.kernel, mesh=mesh, out_type=tuple(out_type),
                       scratch_types=scratch)
    def k(h_hbm, src_hbm, dst_hbm, *refs):
        if with_cnt:
            (agg_out, cnt_out, si_v, di_v, rows_v, zer_v, sem_i,
             sg0, sg1, ss0, ss1, acc, sc0, sc1, ones_v, z16_v, cacc) = refs
            sem_c = (sc0, sc1)
        else:
            (agg_out, si_v, di_v, rows_v, zer_v, sem_i,
             sg0, sg1, ss0, ss1, acc) = refs
        sem_g = (sg0, sg1)
        sem_s = (ss0, ss1)
        c = lax.axis_index("c")
        s = lax.axis_index("s")
        wid = c * NS + s

        zero16 = jnp.zeros((16,), jnp.float32)

        def zrow(i, carry):
            for jj in range(D // 16):
                zer_v[i, jj * 16:(jj + 1) * 16] = zero16
            return carry
        lax.fori_loop(0, ZCH, zrow, 0)
        for t in range(RPW // ZCH):
            pltpu.sync_copy(zer_v, acc.at[pl.ds(s * RPW + t * ZCH, ZCH)])
        if with_cnt:
            one16 = jnp.ones((16,), jnp.float32)
            for jj in range(RPW // 16):
                z16_v[jj * 16:(jj + 1) * 16] = zero16
            for jj in range(128 // 16):
                ones_v[jj * 16:(jj + 1) * 16] = one16
            pltpu.sync_copy(z16_v, cacc.at[pl.ds(s * RPW, RPW)])
        plsc.subcore_barrier()

        NG = NCHUNK // 8
        gat = [None, None]
        sca = [None, None]
        idx_d = []

        def fire_idx(g):
            bg = g % 2
            return [
                pltpu.async_copy(src_hbm.at[wid, pl.ds(g * 8, 8)],
                                 si_v.at[bg], sem_i),
                pltpu.async_copy(dst_hbm.at[wid, pl.ds(g * 8, 8)],
                                 di_v.at[bg], sem_i),
            ]

        idx_d = fire_idx(0)

        def start_gather(j):
            g, jj = divmod(j, 8)
            br = j % 2
            gat[br] = pltpu.async_copy(
                h_hbm.at[si_v.at[g % 2, jj]], rows_v.at[br], sem_g[br])

        def start_scatter(j):
            g, jj = divmod(j, 8)
            br = j % 2
            dsc = [pltpu.async_copy(rows_v.at[br],
                                    acc.at[di_v.at[g % 2, jj]],
                                    sem_s[br], add=True)]
            if with_cnt:
                dsc.append(pltpu.async_copy(ones_v.at[pl.ds(0, CH)],
                                            cacc.at[di_v.at[g % 2, jj]],
                                            sem_c[br], add=True))
            sca[br] = dsc

        for g in range(NG):
            if g == 0:
                for dd in idx_d:
                    dd.wait()
            for jj in range(8):
                j = g * 8 + jj
                br = j % 2
                if sca[br] is not None:
                    for dd in sca[br]:
                        dd.wait()
                    sca[br] = None
                if g > 0 and jj == 0:
                    for dd in idx_d:
                        dd.wait()
                    idx_d = []
                start_gather(j)
                if jj == 1 and g + 1 < NG:
                    idx_d = fire_idx(g + 1)
                if j >= 1:
                    gat[1 - br].wait()
                    start_scatter(j - 1)
        last = NCHUNK - 1
        gat[last % 2].wait()
        start_scatter(last)
        for bb in range(2):
            if sca[bb] is not None:
                for dd in sca[bb]:
                    dd.wait()

        plsc.subcore_barrier()
        pltpu.sync_copy(acc.at[pl.ds(s * RPW, RPW)],
                        agg_out.at[c, pl.ds(s * RPW, RPW)])
        if with_cnt:
            pltpu.sync_copy(cacc.at[pl.ds(s * RPW, RPW)],
                            cnt_out.at[c, pl.ds(s * RPW, RPW)])

    res = k(h, srcr, dstr)
    return res if with_cnt else res[0]


def _tc_stage(agg, cnt, h, Wl, Wr, b):
    BR = 1000

    def body(a_ref, c_ref, h_ref, wl_ref, wr_ref, b_ref, o_ref):
        a = a_ref[0] + a_ref[1]
        deg = c_ref[0] + c_ref[1]
        mean = a * (1.0 / jnp.maximum(deg, 1.0))
        o_ref[...] = (
            jnp.dot(mean, wl_ref[...], preferred_element_type=jnp.float32)
            + jnp.dot(h_ref[...], wr_ref[...], preferred_element_type=jnp.float32)
            + b_ref[...])

    return pl.pallas_call(
        body,
        grid=(N // BR,),
        in_specs=[
            pl.BlockSpec((NC, BR, D), lambda i: (0, i, 0)),
            pl.BlockSpec((NC, BR, 1), lambda i: (0, i, 0)),
            pl.BlockSpec((BR, D), lambda i: (i, 0)),
            pl.BlockSpec((D, D), lambda i: (0, 0)),
            pl.BlockSpec((D, D), lambda i: (0, 0)),
            pl.BlockSpec((1, D), lambda i: (0, 0)),
        ],
        out_specs=pl.BlockSpec((BR, D), lambda i: (i, 0)),
        out_shape=jax.ShapeDtypeStruct((N, D), jnp.float32),
    )(agg, cnt.reshape(NC, NP_, 1), h, Wl, Wr, b.reshape(1, D))


def _mlp_head(hr, Wm1, bm1, Wm2, bm2):
    def body(h_ref, w1_ref, b1_ref, w2_ref, b2_ref, o_ref):
        z = jnp.maximum(
            jnp.dot(h_ref[...], w1_ref[...], preferred_element_type=jnp.float32)
            + b1_ref[...], 0.0)
        o_ref[...] = (jnp.dot(z, w2_ref[...], preferred_element_type=jnp.float32)
                      + b2_ref[...])

    return pl.pallas_call(
        body,
        out_shape=jax.ShapeDtypeStruct((B, OUT), jnp.float32),
    )(hr, Wm1, bm1.reshape(1, MLP_H), Wm2, bm2.reshape(1, OUT))


def kernel(x, edge_index, Wl0, Wr0, b0, Wl1, Wr1, b1, Wl2, Wr2, b2,
           Wm1, bm1, Wm2, bm2):
    ei = edge_index.astype(jnp.int32)
    srcr = ei[0].reshape(NW, NCHUNK, CH)
    dstr = ei[1].reshape(NW, NCHUNK, CH)

    agg1, cnt = _sc_agg(x, srcr, dstr, with_cnt=True)
    h1 = _tc_stage(agg1, cnt, x, Wl0, Wr0, b0)
    agg2 = _sc_agg(h1, srcr, dstr, with_cnt=False)
    h2 = _tc_stage(agg2, cnt, h1, Wl1, Wr1, b1)
    agg3 = _sc_agg(h2, srcr, dstr, with_cnt=False)
    h3 = _tc_stage(agg3, cnt, h2, Wl2, Wr2, b2)

    h3r = h3.reshape(B, N // B, D)[:, 0, :]
    return _mlp_head(h3r, Wm1, bm1, Wm2, bm2)


# speedup vs baseline: 12.0047x; 1.0027x over previous
import functools

import jax
import jax.numpy as jnp
from jax import lax
from jax.experimental import pallas as pl
from jax.experimental.pallas import tpu as pltpu
from jax.experimental.pallas import tpu_sc as plsc

N = 10000
E = 320000
D = 128
B = 100
MLP_H = 256
OUT = 64

NC = 2
NS = 16
NW = NC * NS
CH = 128
NCHUNK = 80
EPT = NCHUNK * CH
E_PAD = NW * EPT
NG = NCHUNK // 8
NP_ = 10240
RPW = NP_ // NS
ZCH = 32
RING = 1280
CCAP = 1152


def _sc_l1(h, srcp, dstp):
    mesh = plsc.VectorSubcoreMesh(core_axis_name="c", subcore_axis_name="s")

    @functools.partial(
        pl.kernel, mesh=mesh,
        compiler_params=pltpu.CompilerParams(needs_layout_passes=False),
        out_type=(jax.ShapeDtypeStruct((NC, NP_, D), jnp.float32),
                  jax.ShapeDtypeStruct((NC, NP_), jnp.float32),
                  jax.ShapeDtypeStruct((NC, NP_), jnp.float32)),
        scratch_types=[
            pltpu.VMEM((2, 8, CH), jnp.int32),
            pltpu.VMEM((2, 8, CH), jnp.int32),
            pltpu.VMEM((2, CH, D), jnp.float32),
            pltpu.VMEM((ZCH, D), jnp.float32),
            pltpu.VMEM((RING,), jnp.int32),
            pltpu.VMEM((1, CH), jnp.int32),
            pltpu.VMEM((CH,), jnp.float32),
            pltpu.VMEM((RPW,), jnp.float32),
            pltpu.SemaphoreType.DMA,
            pltpu.SemaphoreType.DMA,
            pltpu.SemaphoreType.DMA,
            pltpu.SemaphoreType.DMA,
            pltpu.SemaphoreType.DMA,
            pltpu.SemaphoreType.DMA,
            pltpu.SemaphoreType.DMA,
            pltpu.VMEM_SHARED((NP_, D), jnp.float32),
            pltpu.VMEM_SHARED((NP_,), jnp.float32),
            pltpu.VMEM_SHARED((NP_,), jnp.float32),
        ])
    def k(h_hbm, src_hbm, dst_hbm, agg_out, cnt_out, mask_out,
          si_v, di_v, rows_v, zer_v, csrc_v, stage_v, ones_v, z16_v,
          sem_i, sg0, sg1, ss0, ss1, sc0, sc1, acc, cacc, macc):
        sem_g = (sg0, sg1)
        sem_s = (ss0, ss1)
        sem_c = (sc0, sc1)
        c = lax.axis_index("c")
        s = lax.axis_index("s")
        wid = c * NS + s

        zero16 = jnp.zeros((16,), jnp.float32)
        one16 = jnp.ones((16,), jnp.float32)

        def zrow(i, carry):
            for jj in range(D // 16):
                zer_v[i, jj * 16:(jj + 1) * 16] = zero16
            return carry
        lax.fori_loop(0, ZCH, zrow, 0)
        for t in range(RPW // ZCH):
            pltpu.sync_copy(zer_v, acc.at[pl.ds(s * RPW + t * ZCH, ZCH)])
        for jj in range(RPW // 16):
            z16_v[jj * 16:(jj + 1) * 16] = zero16
        for jj in range(CH // 16):
            ones_v[jj * 16:(jj + 1) * 16] = one16
        pltpu.sync_copy(z16_v, cacc.at[pl.ds(s * RPW, RPW)])
        pltpu.sync_copy(z16_v, macc.at[pl.ds(s * RPW, RPW)])
        plsc.subcore_barrier()

        gat = [None, None]
        sca = [None, None]

        def fire_idx(g):
            bg = g % 2
            return [
                pltpu.async_copy(src_hbm.at[wid, pl.ds(g * 8, 8)],
                                 si_v.at[bg], sem_i),
                pltpu.async_copy(dst_hbm.at[wid, pl.ds(g * 8, 8)],
                                 di_v.at[bg], sem_i),
            ]

        idx_d = fire_idx(0)

        def start_gather(j):
            g, jj = divmod(j, 8)
            br = j % 2
            gat[br] = pltpu.async_copy(
                h_hbm.at[si_v.at[g % 2, jj]], rows_v.at[br], sem_g[br])

        def start_scatter(j):
            g, jj = divmod(j, 8)
            br = j % 2
            sca[br] = [
                pltpu.async_copy(rows_v.at[br], acc.at[di_v.at[g % 2, jj]],
                                 sem_s[br], add=True),
                pltpu.async_copy(ones_v, cacc.at[di_v.at[g % 2, jj]],
                                 sem_c[br], add=True),
            ]

        iot = lax.iota(jnp.int32, 16)
        trash_mask_dst = 10001 + 2 * iot
        mBIG = jnp.int32(1 << 30)

        def mask_chunk(k2, carry):
            for kk in range(8):
                vv = csrc_v[pl.ds(k2 * CH + kk * 16, 16)]
                stage_v[0, kk * 16:(kk + 1) * 16] = vv
            pltpu.sync_copy(ones_v, macc.at[stage_v.at[0]], add=True)
            return carry

        def mask_compact(bg, p):
            def cvec(v, pp):
                jj = v // 8
                kk = v % 8
                sv = si_v[bg, jj, pl.ds(kk * 16, 16)]
                dv = di_v[bg, jj, pl.ds(kk * 16, 16)]
                m = lax.rem(dv, 100) == 0
                key = jnp.where(m, sv, mBIG)
                sk, _ = plsc.sort_key_val(key, key)
                csrc_v[pl.ds(pp[0], 16)] = sk
                return pp + plsc.all_reduce_population_count(m)
            p = lax.fori_loop(0, 64, cvec, p)
            ps = p[0]
            nfull = ps // CH
            lax.fori_loop(0, nfull, mask_chunk, 0)
            for kk in range(8):
                vv = csrc_v[pl.ds(nfull * CH + kk * 16, 16)]
                csrc_v[pl.ds(kk * 16, 16)] = vv
            return jnp.zeros((16,), jnp.int32) + (ps - nfull * CH)

        def mask_finish(p):
            ps = p[0]
            for kk in range(8):
                csrc_v[pl.ds(ps + kk * 16, 16)] = trash_mask_dst
            lax.fori_loop(0, (ps + CH - 1) // CH, mask_chunk, 0)

        mp = jnp.zeros((16,), jnp.int32)
        for g in range(NG):
            for jj in range(8):
                j = g * 8 + jj
                br = j % 2
                if sca[br] is not None:
                    for dd in sca[br]:
                        dd.wait()
                    sca[br] = None
                if jj == 0:
                    for dd in idx_d:
                        dd.wait()
                    idx_d = []
                    mp = mask_compact(g % 2, mp)
                start_gather(j)
                if jj == 1 and g + 1 < NG:
                    idx_d = fire_idx(g + 1)
                if j >= 1:
                    gat[1 - br].wait()
                    start_scatter(j - 1)
        last = NCHUNK - 1
        gat[last % 2].wait()
        start_scatter(last)
        mask_finish(mp)
        for bb in range(2):
            if sca[bb] is not None:
                for dd in sca[bb]:
                    dd.wait()

        plsc.subcore_barrier()
        pltpu.sync_copy(acc.at[pl.ds(s * RPW, RPW)],
                        agg_out.at[c, pl.ds(s * RPW, RPW)])
        pltpu.sync_copy(cacc.at[pl.ds(s * RPW, RPW)],
                        cnt_out.at[c, pl.ds(s * RPW, RPW)])
        pltpu.sync_copy(macc.at[pl.ds(s * RPW, RPW)],
                        mask_out.at[c, pl.ds(s * RPW, RPW)])

    return k(h, srcp, dstp)


def _sc_pruned(h, srcp, dstp, mask, root_mode):
    mesh = plsc.VectorSubcoreMesh(core_axis_name="c", subcore_axis_name="s")
    AROWS = 128 if root_mode else NP_
    arpw = AROWS // NS
    scratch = [
        pltpu.VMEM((2, 8, CH), jnp.int32),
        pltpu.VMEM((2, 8, CH), jnp.int32),
        pltpu.VMEM((CH, D), jnp.float32),
        pltpu.VMEM((ZCH, D), jnp.float32),
        pltpu.VMEM((RING,), jnp.int32),
        pltpu.VMEM((1, CH), jnp.int32),
        pltpu.VMEM((1, CH), jnp.int32),
        pltpu.SemaphoreType.DMA,
        pltpu.SemaphoreType.DMA,
        pltpu.VMEM_SHARED((AROWS, D), jnp.float32),
    ]
    if not root_mode:
        scratch.append(pltpu.VMEM((NP_,), jnp.int32))

    @functools.partial(
        pl.kernel, mesh=mesh,
        compiler_params=pltpu.CompilerParams(needs_layout_passes=False),
        out_type=(jax.ShapeDtypeStruct((NC, AROWS, D), jnp.float32),),
        scratch_types=scratch)
    def k(*args):
        if root_mode:
            (h_hbm, src_hbm, dst_hbm, agg_out,
             si_v, di_v, rows_v, zer_v, csrc_v, gidx_v, stage_v,
             sem_i, sem_g, acc) = args
            mask_v = None
        else:
            (h_hbm, src_hbm, dst_hbm, mask_hbm, agg_out,
             si_v, di_v, rows_v, zer_v, csrc_v, gidx_v, stage_v,
             sem_i, sem_g, acc, mask_v) = args
        c = lax.axis_index("c")
        s = lax.axis_index("s")
        wid = c * NS + s

        zero16 = jnp.zeros((16,), jnp.float32)

        def zrow(i, carry):
            for jj in range(D // 16):
                zer_v[i, jj * 16:(jj + 1) * 16] = zero16
            return carry
        lax.fori_loop(0, ZCH, zrow, 0)
        if root_mode:
            pltpu.sync_copy(zer_v.at[pl.ds(0, arpw)],
                            acc.at[pl.ds(s * arpw, arpw)])
        else:
            for t in range(RPW // ZCH):
                pltpu.sync_copy(zer_v, acc.at[pl.ds(s * RPW + t * ZCH, ZCH)])
            pltpu.sync_copy(mask_hbm, mask_v)
        plsc.subcore_barrier()

        def fire_idx(g):
            bg = g % 2
            return [
                pltpu.async_copy(src_hbm.at[wid, pl.ds(g * 8, 8)],
                                 si_v.at[bg], sem_i),
                pltpu.async_copy(dst_hbm.at[wid, pl.ds(g * 8, 8)],
                                 di_v.at[bg], sem_i),
            ]

        iot = lax.iota(jnp.int32, 16)
        trash_src = lax.rem(iot * 523, 9973)
        trash_dst = (100 + iot) if root_mode else (10001 + 2 * iot)
        trash_pk = trash_src * 16384 + trash_dst
        BIG = jnp.int32(1 << 30)

        def proc_chunk(k2, carry):
            for kk in range(8):
                pk = csrc_v[pl.ds(k2 * CH + kk * 16, 16)]
                gidx_v[0, kk * 16:(kk + 1) * 16] = pk >> 14
                stage_v[0, kk * 16:(kk + 1) * 16] = pk & 16383
            pltpu.async_copy(h_hbm.at[gidx_v.at[0]], rows_v, sem_g).wait()
            pltpu.sync_copy(rows_v, acc.at[stage_v.at[0]], add=True)
            return carry

        def compact_group(bg, p):
            def cvec(v, pp):
                jj = v // 8
                kk = v % 8
                sv = si_v[bg, jj, pl.ds(kk * 16, 16)]
                dv = di_v[bg, jj, pl.ds(kk * 16, 16)]
                if root_mode:
                    m = lax.rem(dv, 100) == 0
                    nd = dv // 100
                else:
                    mv = plsc.load_gather(mask_v, [dv])
                    m = mv != 0
                    nd = dv
                key = jnp.where(m, (sv << 14) + nd, BIG)
                sk, _ = plsc.sort_key_val(key, key)
                csrc_v[pl.ds(pp[0], 16)] = sk
                return pp + plsc.all_reduce_population_count(m)
            p = lax.fori_loop(0, 64, cvec, p)
            ps = p[0]
            nfull = ps // CH
            lax.fori_loop(0, nfull, proc_chunk, 0)
            for kk in range(8):
                vv = csrc_v[pl.ds(nfull * CH + kk * 16, 16)]
                csrc_v[pl.ds(kk * 16, 16)] = vv
            return jnp.zeros((16,), jnp.int32) + (ps - nfull * CH)

        idx_d = fire_idx(0)
        p = jnp.zeros((16,), jnp.int32)
        for g in range(NG):
            for dd in idx_d:
                dd.wait()
            if g + 1 < NG:
                idx_d = fire_idx(g + 1)
            p = compact_group(g % 2, p)
        ps = p[0]
        for kk in range(8):
            csrc_v[pl.ds(ps + kk * 16, 16)] = trash_pk
        lax.fori_loop(0, (ps + CH - 1) // CH, proc_chunk, 0)

        plsc.subcore_barrier()
        pltpu.sync_copy(acc.at[pl.ds(s * arpw, arpw)],
                        agg_out.at[c, pl.ds(s * arpw, arpw)])

    if root_mode:
        return k(h, srcp, dstp)[0]
    return k(h, srcp, dstp, mask)[0]


def _tc_stage(agg, cnt, h, Wl, Wr, b, maskp=None):
    BR = 1000
    with_mask = maskp is not None

    def body(*refs):
        if with_mask:
            (a_ref, c_ref, h_ref, wl_ref, wr_ref, b_ref, m_ref,
             o_ref, mo_ref) = refs
        else:
            a_ref, c_ref, h_ref, wl_ref, wr_ref, b_ref, o_ref = refs
        a = a_ref[0] + a_ref[1]
        deg = c_ref[0] + c_ref[1]
        mean = a * (1.0 / jnp.maximum(deg, 1.0))
        o_ref[...] = (
            jnp.dot(mean, wl_ref[...], preferred_element_type=jnp.float32)
            + jnp.dot(h_ref[...], wr_ref[...], preferred_element_type=jnp.float32)
            + b_ref[...])
        if with_mask:
            node = (jax.lax.broadcasted_iota(jnp.int32, (BR, 1), 0)
                    + pl.program_id(0) * BR)
            needed = ((m_ref[0] + m_ref[1]) > 0.0) | (node % 100 == 0)
            mo_ref[...] = needed.astype(jnp.int32)

    in_specs = [
        pl.BlockSpec((NC, BR, D), lambda i: (0, i, 0)),
        pl.BlockSpec((NC, BR, 1), lambda i: (0, i, 0)),
        pl.BlockSpec((BR, D), lambda i: (i, 0)),
        pl.BlockSpec((D, D), lambda i: (0, 0)),
        pl.BlockSpec((D, D), lambda i: (0, 0)),
        pl.BlockSpec((1, D), lambda i: (0, 0)),
    ]
    out_shape = jax.ShapeDtypeStruct((N, D), jnp.float32)
    out_specs = pl.BlockSpec((BR, D), lambda i: (i, 0))
    args = [agg, cnt.reshape(NC, NP_, 1), h, Wl, Wr, b.reshape(1, D)]
    if with_mask:
        in_specs.append(pl.BlockSpec((NC, BR, 1), lambda i: (0, i, 0)))
        out_shape = [out_shape, jax.ShapeDtypeStruct((NP_, 1), jnp.int32)]
        out_specs = [out_specs, pl.BlockSpec((BR, 1), lambda i: (i, 0))]
        args.append(maskp.reshape(NC, NP_, 1))

    return pl.pallas_call(
        body,
        grid=(N // BR,),
        in_specs=in_specs,
        out_specs=out_specs,
        out_shape=out_shape,
    )(*args)


def _tc_head(aggr, cntr, h2r, Wl2, Wr2, b2, Wm1, bm1, Wm2, bm2):
    def body(a_ref, c_ref, h_ref, wl_ref, wr_ref, b_ref,
             w1_ref, b1_ref, w2_ref, b2_ref, o_ref):
        a = a_ref[0, 0:B, :] + a_ref[1, 0:B, :]
        deg = c_ref[0] + c_ref[1]
        mean = a * (1.0 / jnp.maximum(deg, 1.0))
        h3 = (jnp.dot(mean, wl_ref[...], preferred_element_type=jnp.float32)
              + jnp.dot(h_ref[...], wr_ref[...], preferred_element_type=jnp.float32)
              + b_ref[...])
        z = jnp.maximum(
            jnp.dot(h3, w1_ref[...], preferred_element_type=jnp.float32)
            + b1_ref[...], 0.0)
        o_ref[...] = (jnp.dot(z, w2_ref[...], preferred_element_type=jnp.float32)
                      + b2_ref[...])

    return pl.pallas_call(
        body,
        out_shape=jax.ShapeDtypeStruct((B, OUT), jnp.float32),
    )(aggr, cntr, h2r, Wl2, Wr2, b2.reshape(1, D),
      Wm1, bm1.reshape(1, MLP_H), Wm2, bm2.reshape(1, OUT))


def kernel(x, edge_index, Wl0, Wr0, b0, Wl1, Wr1, b1, Wl2, Wr2, b2,
           Wm1, bm1, Wm2, bm2):
    ei = edge_index.astype(jnp.int32)
    npad = E_PAD - E
    ar = jnp.arange(npad, dtype=jnp.int32)
    pad_src = lax.rem(ar * 13, N)
    pad_dst = 10001 + 2 * lax.rem(ar, 119)
    srcp = jnp.concatenate([ei[0], pad_src]).reshape(NW, NCHUNK, CH)
    dstp = jnp.concatenate([ei[1], pad_dst]).reshape(NW, NCHUNK, CH)

    agg1, cnt, maskp = _sc_l1(x, srcp, dstp)
    h1, mask = _tc_stage(agg1, cnt, x, Wl0, Wr0, b0, maskp=maskp)
    agg2 = _sc_pruned(h1, srcp, dstp, mask.reshape(NP_), root_mode=False)
    h2 = _tc_stage(agg2, cnt, h1, Wl1, Wr1, b1)
    agg3r = _sc_pruned(h2, srcp, dstp, None, root_mode=True)

    h2r = h2.reshape(B, N // B, D)[:, 0, :]
    cntr = cnt[:, :N].reshape(NC, B, N // B)[:, :, 0].reshape(NC, B, 1)
    return _tc_head(agg3r, cntr, h2r, Wl2, Wr2, b2, Wm1, bm1, Wm2, bm2)


# speedup vs baseline: 14.6691x; 1.2220x over previous
import functools

import jax
import jax.numpy as jnp
from jax import lax
from jax.experimental import pallas as pl
from jax.experimental.pallas import tpu as pltpu
from jax.experimental.pallas import tpu_sc as plsc

N = 10000
E = 320000
D = 128
B = 100
MLP_H = 256
OUT = 64

NC = 2
NS = 16
NW = NC * NS
CH = 128
NCHUNK = 80
EPT = NCHUNK * CH
E_PAD = NW * EPT
NG = NCHUNK // 8
NP_ = 10240
RPW = NP_ // NS
ZCH = 32
RING = 1280
CCAP = 1152


def _sc_l1(h, srcp, dstp):
    mesh = plsc.VectorSubcoreMesh(core_axis_name="c", subcore_axis_name="s")

    @functools.partial(
        pl.kernel, mesh=mesh,
        compiler_params=pltpu.CompilerParams(needs_layout_passes=False),
        out_type=(jax.ShapeDtypeStruct((NC, NP_, D), jnp.float32),
                  jax.ShapeDtypeStruct((NC, NP_), jnp.float32),
                  jax.ShapeDtypeStruct((NC, NP_), jnp.float32)),
        scratch_types=[
            pltpu.VMEM((2, 8, CH), jnp.int32),
            pltpu.VMEM((2, 8, CH), jnp.int32),
            pltpu.VMEM((2, CH, D), jnp.float32),
            pltpu.VMEM((ZCH, D), jnp.float32),
            pltpu.VMEM((RING,), jnp.int32),
            pltpu.VMEM((1, CH), jnp.int32),
            pltpu.VMEM((CH,), jnp.float32),
            pltpu.VMEM((RPW,), jnp.float32),
            pltpu.SemaphoreType.DMA,
            pltpu.SemaphoreType.DMA,
            pltpu.SemaphoreType.DMA,
            pltpu.SemaphoreType.DMA,
            pltpu.SemaphoreType.DMA,
            pltpu.SemaphoreType.DMA,
            pltpu.SemaphoreType.DMA,
            pltpu.VMEM_SHARED((NP_, D), jnp.float32),
            pltpu.VMEM_SHARED((NP_,), jnp.float32),
            pltpu.VMEM_SHARED((NP_,), jnp.float32),
        ])
    def k(h_hbm, src_hbm, dst_hbm, agg_out, cnt_out, mask_out,
          si_v, di_v, rows_v, zer_v, csrc_v, stage_v, ones_v, z16_v,
          sem_i, sg0, sg1, ss0, ss1, sc0, sc1, acc, cacc, macc):
        sem_g = (sg0, sg1)
        sem_s = (ss0, ss1)
        sem_c = (sc0, sc1)
        c = lax.axis_index("c")
        s = lax.axis_index("s")
        wid = c * NS + s

        zero16 = jnp.zeros((16,), jnp.float32)
        one16 = jnp.ones((16,), jnp.float32)

        def zrow(i, carry):
            for jj in range(D // 16):
                zer_v[i, jj * 16:(jj + 1) * 16] = zero16
            return carry
        lax.fori_loop(0, ZCH, zrow, 0)
        for t in range(RPW // ZCH):
            pltpu.sync_copy(zer_v, acc.at[pl.ds(s * RPW + t * ZCH, ZCH)])
        for jj in range(RPW // 16):
            z16_v[jj * 16:(jj + 1) * 16] = zero16
        for jj in range(CH // 16):
            ones_v[jj * 16:(jj + 1) * 16] = one16
        pltpu.sync_copy(z16_v, cacc.at[pl.ds(s * RPW, RPW)])
        pltpu.sync_copy(z16_v, macc.at[pl.ds(s * RPW, RPW)])
        plsc.subcore_barrier()

        gat = [None, None]
        sca = [None, None]

        def fire_idx(g):
            bg = g % 2
            return [
                pltpu.async_copy(src_hbm.at[wid, pl.ds(g * 8, 8)],
                                 si_v.at[bg], sem_i),
                pltpu.async_copy(dst_hbm.at[wid, pl.ds(g * 8, 8)],
                                 di_v.at[bg], sem_i),
            ]

        idx_d = fire_idx(0)

        def start_gather(j):
            g, jj = divmod(j, 8)
            br = j % 2
            gat[br] = pltpu.async_copy(
                h_hbm.at[si_v.at[g % 2, jj]], rows_v.at[br], sem_g[br])

        def start_scatter(j):
            g, jj = divmod(j, 8)
            br = j % 2
            sca[br] = [
                pltpu.async_copy(rows_v.at[br], acc.at[di_v.at[g % 2, jj]],
                                 sem_s[br], add=True),
                pltpu.async_copy(ones_v, cacc.at[di_v.at[g % 2, jj]],
                                 sem_c[br], add=True),
            ]

        iot = lax.iota(jnp.int32, 16)
        trash_mask_dst = 10001 + 2 * iot
        mBIG = jnp.int32(1 << 30)

        def mask_chunk(k2, carry):
            for kk in range(8):
                vv = csrc_v[pl.ds(k2 * CH + kk * 16, 16)]
                stage_v[0, kk * 16:(kk + 1) * 16] = vv
            pltpu.sync_copy(ones_v, macc.at[stage_v.at[0]], add=True)
            return carry

        def mask_compact(bg, p):
            def cvec(v, pp):
                jj = v // 8
                kk = v % 8
                sv = si_v[bg, jj, pl.ds(kk * 16, 16)]
                dv = di_v[bg, jj, pl.ds(kk * 16, 16)]
                m = lax.rem(dv, 100) == 0
                key = jnp.where(m, sv, mBIG)
                sk, _ = plsc.sort_key_val(key, key)
                csrc_v[pl.ds(pp[0], 16)] = sk
                return pp + plsc.all_reduce_population_count(m)
            p = lax.fori_loop(0, 64, cvec, p)
            ps = p[0]
            nfull = ps // CH
            lax.fori_loop(0, nfull, mask_chunk, 0)
            for kk in range(8):
                vv = csrc_v[pl.ds(nfull * CH + kk * 16, 16)]
                csrc_v[pl.ds(kk * 16, 16)] = vv
            return jnp.zeros((16,), jnp.int32) + (ps - nfull * CH)

        def mask_finish(p):
            ps = p[0]
            for kk in range(8):
                csrc_v[pl.ds(ps + kk * 16, 16)] = trash_mask_dst
            lax.fori_loop(0, (ps + CH - 1) // CH, mask_chunk, 0)

        mp = jnp.zeros((16,), jnp.int32)
        for g in range(NG):
            for jj in range(8):
                j = g * 8 + jj
                br = j % 2
                if sca[br] is not None:
                    for dd in sca[br]:
                        dd.wait()
                    sca[br] = None
                if jj == 0:
                    for dd in idx_d:
                        dd.wait()
                    idx_d = []
                    mp = mask_compact(g % 2, mp)
                start_gather(j)
                if jj == 1 and g + 1 < NG:
                    idx_d = fire_idx(g + 1)
                if j >= 1:
                    gat[1 - br].wait()
                    start_scatter(j - 1)
        last = NCHUNK - 1
        gat[last % 2].wait()
        start_scatter(last)
        mask_finish(mp)
        for bb in range(2):
            if sca[bb] is not None:
                for dd in sca[bb]:
                    dd.wait()

        plsc.subcore_barrier()
        pltpu.sync_copy(acc.at[pl.ds(s * RPW, RPW)],
                        agg_out.at[c, pl.ds(s * RPW, RPW)])
        pltpu.sync_copy(cacc.at[pl.ds(s * RPW, RPW)],
                        cnt_out.at[c, pl.ds(s * RPW, RPW)])
        pltpu.sync_copy(macc.at[pl.ds(s * RPW, RPW)],
                        mask_out.at[c, pl.ds(s * RPW, RPW)])

    return k(h, srcp, dstp)


def _sc_pruned(h, srcp, dstp, mask, root_mode):
    mesh = plsc.VectorSubcoreMesh(core_axis_name="c", subcore_axis_name="s")
    AROWS = 128 if root_mode else NP_
    arpw = AROWS // NS
    scratch = [
        pltpu.VMEM((2, 8, CH), jnp.int32),
        pltpu.VMEM((2, 8, CH), jnp.int32),
        pltpu.VMEM((CH, D), jnp.float32),
        pltpu.VMEM((ZCH, D), jnp.float32),
        pltpu.VMEM((RING,), jnp.int32),
        pltpu.VMEM((1, CH), jnp.int32),
        pltpu.VMEM((1, CH), jnp.int32),
        pltpu.SemaphoreType.DMA,
        pltpu.SemaphoreType.DMA,
        pltpu.VMEM_SHARED((AROWS, D), jnp.float32),
    ]
    if not root_mode:
        scratch.append(pltpu.VMEM((NP_,), jnp.int32))

    @functools.partial(
        pl.kernel, mesh=mesh,
        compiler_params=pltpu.CompilerParams(needs_layout_passes=False),
        out_type=(jax.ShapeDtypeStruct((NC, AROWS, D), jnp.float32),),
        scratch_types=scratch)
    def k(*args):
        if root_mode:
            (h_hbm, src_hbm, dst_hbm, agg_out,
             si_v, di_v, rows_v, zer_v, csrc_v, gidx_v, stage_v,
             sem_i, sem_g, acc) = args
            mask_v = None
        else:
            (h_hbm, src_hbm, dst_hbm, mask_hbm, agg_out,
             si_v, di_v, rows_v, zer_v, csrc_v, gidx_v, stage_v,
             sem_i, sem_g, acc, mask_v) = args
        c = lax.axis_index("c")
        s = lax.axis_index("s")
        wid = c * NS + s

        zero16 = jnp.zeros((16,), jnp.float32)

        def zrow(i, carry):
            for jj in range(D // 16):
                zer_v[i, jj * 16:(jj + 1) * 16] = zero16
            return carry
        lax.fori_loop(0, ZCH, zrow, 0)
        if root_mode:
            pltpu.sync_copy(zer_v.at[pl.ds(0, arpw)],
                            acc.at[pl.ds(s * arpw, arpw)])
        else:
            for t in range(RPW // ZCH):
                pltpu.sync_copy(zer_v, acc.at[pl.ds(s * RPW + t * ZCH, ZCH)])
            pltpu.sync_copy(mask_hbm, mask_v)
        plsc.subcore_barrier()

        def fire_idx(g):
            bg = g % 2
            return [
                pltpu.async_copy(src_hbm.at[wid, pl.ds(g * 8, 8)],
                                 si_v.at[bg], sem_i),
                pltpu.async_copy(dst_hbm.at[wid, pl.ds(g * 8, 8)],
                                 di_v.at[bg], sem_i),
            ]

        iot = lax.iota(jnp.int32, 16)
        trash_src = lax.rem(iot * 523, 9973)
        trash_dst = (100 + iot) if root_mode else (10001 + 2 * iot)
        trash_pk = trash_src * 16384 + trash_dst
        BIG = jnp.int32(1 << 30)

        def proc_chunk(k2, carry):
            for kk in range(8):
                pk = csrc_v[pl.ds(k2 * CH + kk * 16, 16)]
                gidx_v[0, kk * 16:(kk + 1) * 16] = pk >> 14
                stage_v[0, kk * 16:(kk + 1) * 16] = pk & 16383
            pltpu.async_copy(h_hbm.at[gidx_v.at[0]], rows_v, sem_g).wait()
            pltpu.sync_copy(rows_v, acc.at[stage_v.at[0]], add=True)
            return carry

        def compact_group(bg, p):
            def cvec(v, pp):
                jj = v // 8
                kk = v % 8
                sv = si_v[bg, jj, pl.ds(kk * 16, 16)]
                dv = di_v[bg, jj, pl.ds(kk * 16, 16)]
                if root_mode:
                    m = lax.rem(dv, 100) == 0
                    nd = dv // 100
                else:
                    mv = plsc.load_gather(mask_v, [dv])
                    m = mv != 0
                    nd = dv
                key = jnp.where(m, (sv << 14) + nd, BIG)
                sk, _ = plsc.sort_key_val(key, key)
                csrc_v[pl.ds(pp[0], 16)] = sk
                return pp + plsc.all_reduce_population_count(m)
            p = lax.fori_loop(0, 64, cvec, p)
            ps = p[0]
            nfull = ps // CH
            lax.fori_loop(0, nfull, proc_chunk, 0)
            for kk in range(8):
                vv = csrc_v[pl.ds(nfull * CH + kk * 16, 16)]
                csrc_v[pl.ds(kk * 16, 16)] = vv
            return jnp.zeros((16,), jnp.int32) + (ps - nfull * CH)

        idx_d = fire_idx(0)
        p = jnp.zeros((16,), jnp.int32)
        for g in range(NG):
            for dd in idx_d:
                dd.wait()
            if g + 1 < NG:
                idx_d = fire_idx(g + 1)
            p = compact_group(g % 2, p)
        ps = p[0]
        for kk in range(8):
            csrc_v[pl.ds(ps + kk * 16, 16)] = trash_pk
        lax.fori_loop(0, (ps + CH - 1) // CH, proc_chunk, 0)

        plsc.subcore_barrier()
        pltpu.sync_copy(acc.at[pl.ds(s * arpw, arpw)],
                        agg_out.at[c, pl.ds(s * arpw, arpw)])

    if root_mode:
        return k(h, srcp, dstp)[0]
    return k(h, srcp, dstp, mask)[0]


def _tc_stage(agg, cnt, h, Wl, Wr, b, maskp=None):
    BR = 1000
    with_mask = maskp is not None

    def body(*refs):
        if with_mask:
            (a_ref, c_ref, h_ref, wl_ref, wr_ref, b_ref, m_ref,
             o_ref, mo_ref) = refs
        else:
            a_ref, c_ref, h_ref, wl_ref, wr_ref, b_ref, o_ref = refs
        a = a_ref[0] + a_ref[1]
        deg = c_ref[0] + c_ref[1]
        mean = a * (1.0 / jnp.maximum(deg, 1.0))
        o_ref[...] = (
            jnp.dot(mean, wl_ref[...], preferred_element_type=jnp.float32)
            + jnp.dot(h_ref[...], wr_ref[...], preferred_element_type=jnp.float32)
            + b_ref[...])
        if with_mask:
            node = (jax.lax.broadcasted_iota(jnp.int32, (BR, 1), 0)
                    + pl.program_id(0) * BR)
            needed = ((m_ref[0] + m_ref[1]) > 0.0) | (node % 100 == 0)
            mo_ref[...] = needed.astype(jnp.int32)

    in_specs = [
        pl.BlockSpec((NC, BR, D), lambda i: (0, i, 0)),
        pl.BlockSpec((NC, BR, 1), lambda i: (0, i, 0)),
        pl.BlockSpec((BR, D), lambda i: (i, 0)),
        pl.BlockSpec((D, D), lambda i: (0, 0)),
        pl.BlockSpec((D, D), lambda i: (0, 0)),
        pl.BlockSpec((1, D), lambda i: (0, 0)),
    ]
    out_shape = jax.ShapeDtypeStruct((N, D), jnp.float32)
    out_specs = pl.BlockSpec((BR, D), lambda i: (i, 0))
    args = [agg, cnt.reshape(NC, NP_, 1), h, Wl, Wr, b.reshape(1, D)]
    if with_mask:
        in_specs.append(pl.BlockSpec((NC, BR, 1), lambda i: (0, i, 0)))
        out_shape = [out_shape, jax.ShapeDtypeStruct((N, 1), jnp.int32)]
        out_specs = [out_specs, pl.BlockSpec((BR, 1), lambda i: (i, 0))]
        args.append(maskp.reshape(NC, NP_, 1))

    return pl.pallas_call(
        body,
        grid=(N // BR,),
        in_specs=in_specs,
        out_specs=out_specs,
        out_shape=out_shape,
    )(*args)


def _tc_head(aggr, cntr, h2r, Wl2, Wr2, b2, Wm1, bm1, Wm2, bm2):
    def body(a_ref, c_ref, h_ref, wl_ref, wr_ref, b_ref,
             w1_ref, b1_ref, w2_ref, b2_ref, o_ref):
        a = a_ref[0, 0:B, :] + a_ref[1, 0:B, :]
        deg = c_ref[0] + c_ref[1]
        mean = a * (1.0 / jnp.maximum(deg, 1.0))
        h3 = (jnp.dot(mean, wl_ref[...], preferred_element_type=jnp.float32)
              + jnp.dot(h_ref[...], wr_ref[...], preferred_element_type=jnp.float32)
              + b_ref[...])
        z = jnp.maximum(
            jnp.dot(h3, w1_ref[...], preferred_element_type=jnp.float32)
            + b1_ref[...], 0.0)
        o_ref[...] = (jnp.dot(z, w2_ref[...], preferred_element_type=jnp.float32)
                      + b2_ref[...])

    return pl.pallas_call(
        body,
        out_shape=jax.ShapeDtypeStruct((B, OUT), jnp.float32),
    )(aggr, cntr, h2r, Wl2, Wr2, b2.reshape(1, D),
      Wm1, bm1.reshape(1, MLP_H), Wm2, bm2.reshape(1, OUT))


def kernel(x, edge_index, Wl0, Wr0, b0, Wl1, Wr1, b1, Wl2, Wr2, b2,
           Wm1, bm1, Wm2, bm2):
    ei = edge_index.astype(jnp.int32)
    npad = E_PAD - E
    ar = jnp.arange(npad, dtype=jnp.int32)
    pad_src = lax.rem(ar * 13, N)
    pad_dst = 10001 + 2 * lax.rem(ar, 119)
    srcp = jnp.concatenate([ei[0], pad_src]).reshape(NW, NCHUNK, CH)
    dstp = jnp.concatenate([ei[1], pad_dst]).reshape(NW, NCHUNK, CH)

    agg1, cnt, maskp = _sc_l1(x, srcp, dstp)
    h1, mask = _tc_stage(agg1, cnt, x, Wl0, Wr0, b0, maskp=maskp)
    maskf = jnp.concatenate([mask.reshape(N),
                             jnp.zeros((NP_ - N,), jnp.int32)])
    agg2 = _sc_pruned(h1, srcp, dstp, maskf, root_mode=False)
    h2 = _tc_stage(agg2, cnt, h1, Wl1, Wr1, b1)
    agg3r = _sc_pruned(h2, srcp, dstp, None, root_mode=True)

    h2r = h2.reshape(B, N // B, D)[:, 0, :]
    cntr = cnt[:, :N].reshape(NC, B, N // B)[:, :, 0].reshape(NC, B, 1)
    return _tc_head(agg3r, cntr, h2r, Wl2, Wr2, b2, Wm1, bm1, Wm2, bm2)


# speedup vs baseline: 16.0187x; 1.0920x over previous
import functools

import jax
import jax.numpy as jnp
from jax import lax
from jax.experimental import pallas as pl
from jax.experimental.pallas import tpu as pltpu
from jax.experimental.pallas import tpu_sc as plsc

N = 10000
E = 320000
D = 128
B = 100
MLP_H = 256
OUT = 64

NC = 2
NS = 16
NW = NC * NS
CH = 128
NCHUNK = 80
EPT = NCHUNK * CH
E_PAD = NW * EPT
NG = NCHUNK // 8
NP_ = 10240
RPW = NP_ // NS
ZCH = 32
RING = 1280
EPC = EPT + CH


def _sc_l1(h, srcp, dstp):
    mesh = plsc.VectorSubcoreMesh(core_axis_name="c", subcore_axis_name="s")

    @functools.partial(
        pl.kernel, mesh=mesh,
        compiler_params=pltpu.CompilerParams(needs_layout_passes=False),
        out_type=(jax.ShapeDtypeStruct((NC, NP_, D), jnp.float32),
                  jax.ShapeDtypeStruct((NC, NP_), jnp.float32),
                  jax.ShapeDtypeStruct((NC, NP_), jnp.float32),
                  jax.ShapeDtypeStruct((NW, EPC), jnp.int32),
                  jax.ShapeDtypeStruct((NW, CH), jnp.int32)),
        scratch_types=[
            pltpu.VMEM((2, 8, CH), jnp.int32),
            pltpu.VMEM((2, 8, CH), jnp.int32),
            pltpu.VMEM((2, CH, D), jnp.float32),
            pltpu.VMEM((ZCH, D), jnp.float32),
            pltpu.VMEM((RING,), jnp.int32),
            pltpu.VMEM((1, CH), jnp.int32),
            pltpu.VMEM((CH,), jnp.float32),
            pltpu.VMEM((RPW,), jnp.float32),
            pltpu.SemaphoreType.DMA,
            pltpu.SemaphoreType.DMA,
            pltpu.SemaphoreType.DMA,
            pltpu.SemaphoreType.DMA,
            pltpu.SemaphoreType.DMA,
            pltpu.SemaphoreType.DMA,
            pltpu.SemaphoreType.DMA,
            pltpu.VMEM_SHARED((NP_, D), jnp.float32),
            pltpu.VMEM_SHARED((NP_,), jnp.float32),
            pltpu.VMEM_SHARED((NP_,), jnp.float32),
        ])
    def k(h_hbm, src_hbm, dst_hbm, agg_out, cnt_out, mask_out,
          epack_out, ecnt_out,
          si_v, di_v, rows_v, zer_v, csrc_v, stage_v, ones_v, z16_v,
          sem_i, sg0, sg1, ss0, ss1, sc0, sc1, acc, cacc, macc):
        sem_g = (sg0, sg1)
        sem_s = (ss0, ss1)
        sem_c = (sc0, sc1)
        c = lax.axis_index("c")
        s = lax.axis_index("s")
        wid = c * NS + s

        zero16 = jnp.zeros((16,), jnp.float32)
        one16 = jnp.ones((16,), jnp.float32)

        def zrow(i, carry):
            for jj in range(D // 16):
                zer_v[i, jj * 16:(jj + 1) * 16] = zero16
            return carry
        lax.fori_loop(0, ZCH, zrow, 0)
        for t in range(RPW // ZCH):
            pltpu.sync_copy(zer_v, acc.at[pl.ds(s * RPW + t * ZCH, ZCH)])
        for jj in range(RPW // 16):
            z16_v[jj * 16:(jj + 1) * 16] = zero16
        for jj in range(CH // 16):
            ones_v[jj * 16:(jj + 1) * 16] = one16
        pltpu.sync_copy(z16_v, cacc.at[pl.ds(s * RPW, RPW)])
        pltpu.sync_copy(z16_v, macc.at[pl.ds(s * RPW, RPW)])
        plsc.subcore_barrier()

        gat = [None, None]
        sca = [None, None]

        def fire_idx(g):
            bg = g % 2
            return [
                pltpu.async_copy(src_hbm.at[wid, pl.ds(g * 8, 8)],
                                 si_v.at[bg], sem_i),
                pltpu.async_copy(dst_hbm.at[wid, pl.ds(g * 8, 8)],
                                 di_v.at[bg], sem_i),
            ]

        idx_d = fire_idx(0)

        def start_gather(j):
            g, jj = divmod(j, 8)
            br = j % 2
            gat[br] = pltpu.async_copy(
                h_hbm.at[si_v.at[g % 2, jj]], rows_v.at[br], sem_g[br])

        def start_scatter(j):
            g, jj = divmod(j, 8)
            br = j % 2
            sca[br] = [
                pltpu.async_copy(rows_v.at[br], acc.at[di_v.at[g % 2, jj]],
                                 sem_s[br], add=True),
                pltpu.async_copy(ones_v, cacc.at[di_v.at[g % 2, jj]],
                                 sem_c[br], add=True),
            ]

        iot = lax.iota(jnp.int32, 16)
        trash_pk = ((10001 + 2 * iot) << 14) + (100 + iot)
        mBIG = jnp.int32(1 << 30)

        def mask_chunk(k2, gk):
            for kk in range(8):
                vv = csrc_v[pl.ds(k2 * CH + kk * 16, 16)]
                stage_v[0, kk * 16:(kk + 1) * 16] = vv >> 14
            pltpu.sync_copy(ones_v, macc.at[stage_v.at[0]], add=True)
            pltpu.sync_copy(csrc_v.at[pl.ds(k2 * CH, CH)],
                            epack_out.at[wid, pl.ds(gk * CH, CH)])
            return gk + 1

        def mask_compact(bg, p, gk):
            def cvec(v, pp):
                jj = v // 8
                kk = v % 8
                sv = si_v[bg, jj, pl.ds(kk * 16, 16)]
                dv = di_v[bg, jj, pl.ds(kk * 16, 16)]
                m = lax.rem(dv, 100) == 0
                key = jnp.where(m, (sv << 14) + dv, mBIG)
                sk, _ = plsc.sort_key_val(key, key)
                csrc_v[pl.ds(pp[0], 16)] = sk
                return pp + plsc.all_reduce_population_count(m)
            p = lax.fori_loop(0, 64, cvec, p)
            ps = p[0]
            nfull = ps // CH
            gk = lax.fori_loop(0, nfull, mask_chunk, gk)
            for kk in range(8):
                vv = csrc_v[pl.ds(nfull * CH + kk * 16, 16)]
                csrc_v[pl.ds(kk * 16, 16)] = vv
            return jnp.zeros((16,), jnp.int32) + (ps - nfull * CH), gk

        def mask_finish(p, gk):
            ps = p[0]
            for kk in range(8):
                csrc_v[pl.ds(ps + kk * 16, 16)] = trash_pk
            gk = lax.fori_loop(0, (ps + CH - 1) // CH, mask_chunk, gk)
            for kk in range(8):
                stage_v[0, kk * 16:(kk + 1) * 16] = iot * 0 + gk
            pltpu.sync_copy(stage_v.at[0], ecnt_out.at[wid])

        mp = jnp.zeros((16,), jnp.int32)
        mgk = jnp.int32(0)
        for g in range(NG):
            for jj in range(8):
                j = g * 8 + jj
                br = j % 2
                if sca[br] is not None:
                    for dd in sca[br]:
                        dd.wait()
                    sca[br] = None
                if jj == 0:
                    for dd in idx_d:
                        dd.wait()
                    idx_d = []
                    mp, mgk = mask_compact(g % 2, mp, mgk)
                start_gather(j)
                if jj == 1 and g + 1 < NG:
                    idx_d = fire_idx(g + 1)
                if j >= 1:
                    gat[1 - br].wait()
                    start_scatter(j - 1)
        last = NCHUNK - 1
        gat[last % 2].wait()
        start_scatter(last)
        mask_finish(mp, mgk)
        for bb in range(2):
            if sca[bb] is not None:
                for dd in sca[bb]:
                    dd.wait()

        plsc.subcore_barrier()
        pltpu.sync_copy(acc.at[pl.ds(s * RPW, RPW)],
                        agg_out.at[c, pl.ds(s * RPW, RPW)])
        pltpu.sync_copy(cacc.at[pl.ds(s * RPW, RPW)],
                        cnt_out.at[c, pl.ds(s * RPW, RPW)])
        pltpu.sync_copy(macc.at[pl.ds(s * RPW, RPW)],
                        mask_out.at[c, pl.ds(s * RPW, RPW)])

    return k(h, srcp, dstp)


def _sc_pruned(h, srcp, dstp, mask):
    mesh = plsc.VectorSubcoreMesh(core_axis_name="c", subcore_axis_name="s")

    @functools.partial(
        pl.kernel, mesh=mesh,
        compiler_params=pltpu.CompilerParams(needs_layout_passes=False),
        out_type=(jax.ShapeDtypeStruct((NC, NP_, D), jnp.float32),),
        scratch_types=[
            pltpu.VMEM((2, 8, CH), jnp.int32),
            pltpu.VMEM((2, 8, CH), jnp.int32),
            pltpu.VMEM((CH, D), jnp.float32),
            pltpu.VMEM((ZCH, D), jnp.float32),
            pltpu.VMEM((RING,), jnp.int32),
            pltpu.VMEM((1, CH), jnp.int32),
            pltpu.VMEM((1, CH), jnp.int32),
            pltpu.SemaphoreType.DMA,
            pltpu.SemaphoreType.DMA,
            pltpu.VMEM_SHARED((NP_, D), jnp.float32),
            pltpu.VMEM((NP_,), jnp.int32),
        ])
    def k(h_hbm, src_hbm, dst_hbm, mask_hbm, agg_out,
          si_v, di_v, rows_v, zer_v, csrc_v, gidx_v, stage_v,
          sem_i, sem_g, acc, mask_v):
        c = lax.axis_index("c")
        s = lax.axis_index("s")
        wid = c * NS + s

        zero16 = jnp.zeros((16,), jnp.float32)

        def zrow(i, carry):
            for jj in range(D // 16):
                zer_v[i, jj * 16:(jj + 1) * 16] = zero16
            return carry
        lax.fori_loop(0, ZCH, zrow, 0)
        for t in range(RPW // ZCH):
            pltpu.sync_copy(zer_v, acc.at[pl.ds(s * RPW + t * ZCH, ZCH)])
        pltpu.sync_copy(mask_hbm, mask_v)
        plsc.subcore_barrier()

        def fire_idx(g):
            bg = g % 2
            return [
                pltpu.async_copy(src_hbm.at[wid, pl.ds(g * 8, 8)],
                                 si_v.at[bg], sem_i),
                pltpu.async_copy(dst_hbm.at[wid, pl.ds(g * 8, 8)],
                                 di_v.at[bg], sem_i),
            ]

        iot = lax.iota(jnp.int32, 16)
        trash_src = lax.rem(iot * 523, 9973)
        trash_pk = (trash_src << 14) + (10001 + 2 * iot)
        BIG = jnp.int32(1 << 30)

        def proc_chunk(k2, carry):
            for kk in range(8):
                pk = csrc_v[pl.ds(k2 * CH + kk * 16, 16)]
                gidx_v[0, kk * 16:(kk + 1) * 16] = pk >> 14
                stage_v[0, kk * 16:(kk + 1) * 16] = pk & 16383
            pltpu.async_copy(h_hbm.at[gidx_v.at[0]], rows_v, sem_g).wait()
            pltpu.sync_copy(rows_v, acc.at[stage_v.at[0]], add=True)
            return carry

        def compact_group(bg, p):
            def cvec(v, pp):
                jj = v // 8
                kk = v % 8
                sv = si_v[bg, jj, pl.ds(kk * 16, 16)]
                dv = di_v[bg, jj, pl.ds(kk * 16, 16)]
                mv = plsc.load_gather(mask_v, [dv])
                m = mv != 0
                key = jnp.where(m, (sv << 14) + dv, BIG)
                sk, _ = plsc.sort_key_val(key, key)
                csrc_v[pl.ds(pp[0], 16)] = sk
                return pp + plsc.all_reduce_population_count(m)
            p = lax.fori_loop(0, 64, cvec, p)
            ps = p[0]
            nfull = ps // CH
            lax.fori_loop(0, nfull, proc_chunk, 0)
            for kk in range(8):
                vv = csrc_v[pl.ds(nfull * CH + kk * 16, 16)]
                csrc_v[pl.ds(kk * 16, 16)] = vv
            return jnp.zeros((16,), jnp.int32) + (ps - nfull * CH)

        idx_d = fire_idx(0)
        p = jnp.zeros((16,), jnp.int32)
        for g in range(NG):
            for dd in idx_d:
                dd.wait()
            if g + 1 < NG:
                idx_d = fire_idx(g + 1)
            p = compact_group(g % 2, p)
        ps = p[0]
        for kk in range(8):
            csrc_v[pl.ds(ps + kk * 16, 16)] = trash_pk
        lax.fori_loop(0, (ps + CH - 1) // CH, proc_chunk, 0)

        plsc.subcore_barrier()
        pltpu.sync_copy(acc.at[pl.ds(s * RPW, RPW)],
                        agg_out.at[c, pl.ds(s * RPW, RPW)])

    return k(h, srcp, dstp, mask)[0]


def _sc_root(h2p, epack, ecnt):
    mesh = plsc.VectorSubcoreMesh(core_axis_name="c", subcore_axis_name="s")
    AROWS = 128
    arpw = AROWS // NS

    @functools.partial(
        pl.kernel, mesh=mesh,
        compiler_params=pltpu.CompilerParams(needs_layout_passes=False),
        out_type=(jax.ShapeDtypeStruct((NC, AROWS, D), jnp.float32),),
        scratch_types=[
            pltpu.VMEM((RING,), jnp.int32),
            pltpu.VMEM((CH,), jnp.int32),
            pltpu.VMEM((CH, D), jnp.float32),
            pltpu.VMEM((ZCH, D), jnp.float32),
            pltpu.VMEM((1, CH), jnp.int32),
            pltpu.VMEM((1, CH), jnp.int32),
            pltpu.SemaphoreType.DMA,
            pltpu.SemaphoreType.DMA,
            pltpu.VMEM_SHARED((AROWS, D), jnp.float32),
        ])
    def k(h_hbm, epack_hbm, ecnt_hbm, agg_out,
          pk_v, cnt_v, rows_v, zer_v, gidx_v, stage_v, sem_i, sem_g, acc):
        c = lax.axis_index("c")
        s = lax.axis_index("s")
        wid = c * NS + s

        zero16 = jnp.zeros((16,), jnp.float32)

        def zrow(i, carry):
            for jj in range(D // 16):
                zer_v[i, jj * 16:(jj + 1) * 16] = zero16
            return carry
        lax.fori_loop(0, arpw, zrow, 0)
        pltpu.sync_copy(zer_v.at[pl.ds(0, arpw)],
                        acc.at[pl.ds(s * arpw, arpw)])
        pltpu.sync_copy(ecnt_hbm.at[wid], cnt_v)
        plsc.subcore_barrier()

        nch = cnt_v[0:16][0]

        def proc_chunk(k2, carry):
            pltpu.async_copy(epack_hbm.at[wid, pl.ds(k2 * CH, CH)],
                             pk_v.at[pl.ds(0, CH)], sem_i).wait()
            for kk in range(8):
                pk = pk_v[pl.ds(kk * 16, 16)]
                gidx_v[0, kk * 16:(kk + 1) * 16] = pk >> 14
                stage_v[0, kk * 16:(kk + 1) * 16] = (pk & 16383) // 100
            pltpu.async_copy(h_hbm.at[gidx_v.at[0]], rows_v, sem_g).wait()
            pltpu.sync_copy(rows_v, acc.at[stage_v.at[0]], add=True)
            return carry
        lax.fori_loop(0, nch, proc_chunk, 0)

        plsc.subcore_barrier()
        pltpu.sync_copy(acc.at[pl.ds(s * arpw, arpw)],
                        agg_out.at[c, pl.ds(s * arpw, arpw)])

    return k(h2p, epack, ecnt)[0]


def _tc_stage(agg, cnt, h, Wl, Wr, b, maskp=None):
    BR = 1000
    with_mask = maskp is not None

    def body(*refs):
        if with_mask:
            (a_ref, c_ref, h_ref, wl_ref, wr_ref, b_ref, m_ref,
             o_ref, mo_ref) = refs
        else:
            a_ref, c_ref, h_ref, wl_ref, wr_ref, b_ref, o_ref = refs
        a = a_ref[0] + a_ref[1]
        deg = c_ref[0] + c_ref[1]
        mean = a * (1.0 / jnp.maximum(deg, 1.0))
        o_ref[...] = (
            jnp.dot(mean, wl_ref[...], preferred_element_type=jnp.float32)
            + jnp.dot(h_ref[...], wr_ref[...], preferred_element_type=jnp.float32)
            + b_ref[...])
        if with_mask:
            node = (jax.lax.broadcasted_iota(jnp.int32, (BR, 1), 0)
                    + pl.program_id(0) * BR)
            needed = ((m_ref[0] + m_ref[1]) > 0.0) | (node % 100 == 0)
            mo_ref[...] = needed.astype(jnp.int32)

    in_specs = [
        pl.BlockSpec((NC, BR, D), lambda i: (0, i, 0)),
        pl.BlockSpec((NC, BR, 1), lambda i: (0, i, 0)),
        pl.BlockSpec((BR, D), lambda i: (i, 0)),
        pl.BlockSpec((D, D), lambda i: (0, 0)),
        pl.BlockSpec((D, D), lambda i: (0, 0)),
        pl.BlockSpec((1, D), lambda i: (0, 0)),
    ]
    out_shape = jax.ShapeDtypeStruct((N, D), jnp.float32)
    out_specs = pl.BlockSpec((BR, D), lambda i: (i, 0))
    args = [agg, cnt.reshape(NC, NP_, 1), h, Wl, Wr, b.reshape(1, D)]
    if with_mask:
        in_specs.append(pl.BlockSpec((NC, BR, 1), lambda i: (0, i, 0)))
        out_shape = [out_shape, jax.ShapeDtypeStruct((N, 1), jnp.int32)]
        out_specs = [out_specs, pl.BlockSpec((BR, 1), lambda i: (i, 0))]
        args.append(maskp.reshape(NC, NP_, 1))

    return pl.pallas_call(
        body,
        grid=(N // BR,),
        in_specs=in_specs,
        out_specs=out_specs,
        out_shape=out_shape,
    )(*args)


def _tc_head(aggr, cntr, h2r, Wl2, Wr2, b2, Wm1, bm1, Wm2, bm2):
    def body(a_ref, c_ref, h_ref, wl_ref, wr_ref, b_ref,
             w1_ref, b1_ref, w2_ref, b2_ref, o_ref):
        a = a_ref[0, 0:B, :] + a_ref[1, 0:B, :]
        deg = c_ref[0] + c_ref[1]
        mean = a * (1.0 / jnp.maximum(deg, 1.0))
        h3 = (jnp.dot(mean, wl_ref[...], preferred_element_type=jnp.float32)
              + jnp.dot(h_ref[...], wr_ref[...], preferred_element_type=jnp.float32)
              + b_ref[...])
        z = jnp.maximum(
            jnp.dot(h3, w1_ref[...], preferred_element_type=jnp.float32)
            + b1_ref[...], 0.0)
        o_ref[...] = (jnp.dot(z, w2_ref[...], preferred_element_type=jnp.float32)
                      + b2_ref[...])

    return pl.pallas_call(
        body,
        out_shape=jax.ShapeDtypeStruct((B, OUT), jnp.float32),
    )(aggr, cntr, h2r, Wl2, Wr2, b2.reshape(1, D),
      Wm1, bm1.reshape(1, MLP_H), Wm2, bm2.reshape(1, OUT))


def kernel(x, edge_index, Wl0, Wr0, b0, Wl1, Wr1, b1, Wl2, Wr2, b2,
           Wm1, bm1, Wm2, bm2):
    ei = edge_index.astype(jnp.int32)
    npad = E_PAD - E
    ar = jnp.arange(npad, dtype=jnp.int32)
    pad_src = lax.rem(ar * 13, N)
    pad_dst = 10001 + 2 * lax.rem(ar, 119)
    srcp = jnp.concatenate([ei[0], pad_src]).reshape(NW, NCHUNK, CH)
    dstp = jnp.concatenate([ei[1], pad_dst]).reshape(NW, NCHUNK, CH)

    agg1, cnt, maskp, epack, ecnt = _sc_l1(x, srcp, dstp)
    h1, mask = _tc_stage(agg1, cnt, x, Wl0, Wr0, b0, maskp=maskp)
    maskf = jnp.concatenate([mask.reshape(N),
                             jnp.zeros((NP_ - N,), jnp.int32)])
    agg2 = _sc_pruned(h1, srcp, dstp, maskf)
    h2 = _tc_stage(agg2, cnt, h1, Wl1, Wr1, b1)
    h2p = jnp.concatenate([h2, jnp.zeros((NP_ - N, D), jnp.float32)])
    agg3r = _sc_root(h2p, epack, ecnt)

    h2r = h2.reshape(B, N // B, D)[:, 0, :]
    cntr = cnt[:, :N].reshape(NC, B, N // B)[:, :, 0].reshape(NC, B, 1)
    return _tc_head(agg3r, cntr, h2r, Wl2, Wr2, b2, Wm1, bm1, Wm2, bm2)


# speedup vs baseline: 16.2771x; 1.0161x over previous
import functools

import jax
import jax.numpy as jnp
from jax import lax
from jax.experimental import pallas as pl
from jax.experimental.pallas import tpu as pltpu
from jax.experimental.pallas import tpu_sc as plsc

N = 10000
E = 320000
D = 128
B = 100
MLP_H = 256
OUT = 64

NC = 2
NS = 16
NW = NC * NS
CH = 128
NCHUNK = 80
EPT = NCHUNK * CH
E_PAD = NW * EPT
NG = NCHUNK // 8
NP_ = 10240
RPW = NP_ // NS
ZCH = 32
RING = 1280
EPC = EPT + CH


def _sc_l1(h, srcp, dstp):
    mesh = plsc.VectorSubcoreMesh(core_axis_name="c", subcore_axis_name="s")

    @functools.partial(
        pl.kernel, mesh=mesh,
        compiler_params=pltpu.CompilerParams(needs_layout_passes=False),
        out_type=(jax.ShapeDtypeStruct((NC, NP_, D), jnp.float32),
                  jax.ShapeDtypeStruct((NC, NP_), jnp.float32),
                  jax.ShapeDtypeStruct((NC, NP_), jnp.float32),
                  jax.ShapeDtypeStruct((NW, EPC), jnp.int32),
                  jax.ShapeDtypeStruct((NW, CH), jnp.int32)),
        scratch_types=[
            pltpu.VMEM((2, 8, CH), jnp.int32),
            pltpu.VMEM((2, 8, CH), jnp.int32),
            pltpu.VMEM((2, CH, D), jnp.float32),
            pltpu.VMEM((ZCH, D), jnp.float32),
            pltpu.VMEM((RING,), jnp.int32),
            pltpu.VMEM((1, CH), jnp.int32),
            pltpu.VMEM((CH,), jnp.float32),
            pltpu.VMEM((RPW,), jnp.float32),
            pltpu.SemaphoreType.DMA,
            pltpu.SemaphoreType.DMA,
            pltpu.SemaphoreType.DMA,
            pltpu.SemaphoreType.DMA,
            pltpu.SemaphoreType.DMA,
            pltpu.SemaphoreType.DMA,
            pltpu.SemaphoreType.DMA,
            pltpu.VMEM_SHARED((NP_, D), jnp.float32),
            pltpu.VMEM_SHARED((NP_,), jnp.float32),
            pltpu.VMEM_SHARED((NP_,), jnp.float32),
        ])
    def k(h_hbm, src_hbm, dst_hbm, agg_out, cnt_out, mask_out,
          epack_out, ecnt_out,
          si_v, di_v, rows_v, zer_v, csrc_v, stage_v, ones_v, z16_v,
          sem_i, sg0, sg1, ss0, ss1, sc0, sc1, acc, cacc, macc):
        sem_g = (sg0, sg1)
        sem_s = (ss0, ss1)
        sem_c = (sc0, sc1)
        c = lax.axis_index("c")
        s = lax.axis_index("s")
        wid = c * NS + s

        zero16 = jnp.zeros((16,), jnp.float32)
        one16 = jnp.ones((16,), jnp.float32)

        def zrow(i, carry):
            for jj in range(D // 16):
                zer_v[i, jj * 16:(jj + 1) * 16] = zero16
            return carry
        lax.fori_loop(0, ZCH, zrow, 0)
        for t in range(RPW // ZCH):
            pltpu.sync_copy(zer_v, acc.at[pl.ds(s * RPW + t * ZCH, ZCH)])
        for jj in range(RPW // 16):
            z16_v[jj * 16:(jj + 1) * 16] = zero16
        for jj in range(CH // 16):
            ones_v[jj * 16:(jj + 1) * 16] = one16
        pltpu.sync_copy(z16_v, cacc.at[pl.ds(s * RPW, RPW)])
        pltpu.sync_copy(z16_v, macc.at[pl.ds(s * RPW, RPW)])
        plsc.subcore_barrier()

        gat = [None, None]
        sca = [None, None]

        def fire_idx(g):
            bg = g % 2
            return [
                pltpu.async_copy(src_hbm.at[wid, pl.ds(g * 8, 8)],
                                 si_v.at[bg], sem_i),
                pltpu.async_copy(dst_hbm.at[wid, pl.ds(g * 8, 8)],
                                 di_v.at[bg], sem_i),
            ]

        idx_d = fire_idx(0)

        def start_gather(j):
            g, jj = divmod(j, 8)
            br = j % 2
            gat[br] = pltpu.async_copy(
                h_hbm.at[si_v.at[g % 2, jj]], rows_v.at[br], sem_g[br])

        def start_scatter(j):
            g, jj = divmod(j, 8)
            br = j % 2
            sca[br] = [
                pltpu.async_copy(rows_v.at[br], acc.at[di_v.at[g % 2, jj]],
                                 sem_s[br], add=True),
                pltpu.async_copy(ones_v, cacc.at[di_v.at[g % 2, jj]],
                                 sem_c[br], add=True),
            ]

        iot = lax.iota(jnp.int32, 16)
        trash_pk = ((10001 + 2 * iot) << 14) + (10001 + 2 * iot)
        mBIG = jnp.int32(1 << 30)

        def mask_chunk(k2, gk):
            for kk in range(8):
                vv = csrc_v[pl.ds(k2 * CH + kk * 16, 16)]
                stage_v[0, kk * 16:(kk + 1) * 16] = vv >> 14
            pltpu.sync_copy(ones_v, macc.at[stage_v.at[0]], add=True)
            pltpu.sync_copy(csrc_v.at[pl.ds(k2 * CH, CH)],
                            epack_out.at[wid, pl.ds(gk * CH, CH)])
            return gk + 1

        def mask_compact(bg, p, gk):
            def cvec(v, pp):
                jj = v // 8
                kk = v % 8
                sv = si_v[bg, jj, pl.ds(kk * 16, 16)]
                dv = di_v[bg, jj, pl.ds(kk * 16, 16)]
                m = lax.rem(dv, 100) == 0
                key = jnp.where(m, (sv << 14) + dv, mBIG)
                sk, _ = plsc.sort_key_val(key, key)
                csrc_v[pl.ds(pp[0], 16)] = sk
                return pp + plsc.all_reduce_population_count(m)
            p = lax.fori_loop(0, 64, cvec, p)
            ps = p[0]
            nfull = ps // CH
            gk = lax.fori_loop(0, nfull, mask_chunk, gk)
            for kk in range(8):
                vv = csrc_v[pl.ds(nfull * CH + kk * 16, 16)]
                csrc_v[pl.ds(kk * 16, 16)] = vv
            return jnp.zeros((16,), jnp.int32) + (ps - nfull * CH), gk

        def mask_finish(p, gk):
            ps = p[0]
            for kk in range(8):
                csrc_v[pl.ds(ps + kk * 16, 16)] = trash_pk
            gk = lax.fori_loop(0, (ps + CH - 1) // CH, mask_chunk, gk)
            for kk in range(8):
                stage_v[0, kk * 16:(kk + 1) * 16] = iot * 0 + gk
            pltpu.sync_copy(stage_v.at[0], ecnt_out.at[wid])

        mp = jnp.zeros((16,), jnp.int32)
        mgk = jnp.int32(0)
        for g in range(NG):
            for jj in range(8):
                j = g * 8 + jj
                br = j % 2
                if sca[br] is not None:
                    for dd in sca[br]:
                        dd.wait()
                    sca[br] = None
                if jj == 0:
                    for dd in idx_d:
                        dd.wait()
                    idx_d = []
                    mp, mgk = mask_compact(g % 2, mp, mgk)
                start_gather(j)
                if jj == 1 and g + 1 < NG:
                    idx_d = fire_idx(g + 1)
                if j >= 1:
                    gat[1 - br].wait()
                    start_scatter(j - 1)
        last = NCHUNK - 1
        gat[last % 2].wait()
        start_scatter(last)
        mask_finish(mp, mgk)
        for bb in range(2):
            if sca[bb] is not None:
                for dd in sca[bb]:
                    dd.wait()

        plsc.subcore_barrier()
        pltpu.sync_copy(acc.at[pl.ds(s * RPW, RPW)],
                        agg_out.at[c, pl.ds(s * RPW, RPW)])
        pltpu.sync_copy(cacc.at[pl.ds(s * RPW, RPW)],
                        cnt_out.at[c, pl.ds(s * RPW, RPW)])
        pltpu.sync_copy(macc.at[pl.ds(s * RPW, RPW)],
                        mask_out.at[c, pl.ds(s * RPW, RPW)])

    return k(h, srcp, dstp)


def _sc_pruned(h, srcp, dstp, mask):
    mesh = plsc.VectorSubcoreMesh(core_axis_name="c", subcore_axis_name="s")
    NG2 = NCHUNK // 4

    @functools.partial(
        pl.kernel, mesh=mesh,
        compiler_params=pltpu.CompilerParams(needs_layout_passes=False),
        out_type=(jax.ShapeDtypeStruct((NC, NP_, D), jnp.float32),),
        scratch_types=[
            pltpu.VMEM((2, 4, CH), jnp.int32),
            pltpu.VMEM((2, 4, CH), jnp.int32),
            pltpu.VMEM((2, CH, D), jnp.float32),
            pltpu.VMEM((RING,), jnp.int32),
            pltpu.VMEM((2, CH), jnp.int32),
            pltpu.VMEM((2, CH), jnp.int32),
            pltpu.SemaphoreType.DMA,
            pltpu.SemaphoreType.DMA,
            pltpu.SemaphoreType.DMA,
            pltpu.SemaphoreType.DMA,
            pltpu.SemaphoreType.DMA,
            pltpu.VMEM_SHARED((NP_, D), jnp.float32),
            pltpu.VMEM((NP_,), jnp.int32),
        ])
    def k(h_hbm, src_hbm, dst_hbm, mask_hbm, agg_out,
          si_v, di_v, rows_v, csrc_v, gidx_v, stage_v,
          sem_i, sem_ga, sem_gb, sem_sa, sem_sb, acc, mask_v):
        c = lax.axis_index("c")
        s = lax.axis_index("s")
        wid = c * NS + s

        zero16 = jnp.zeros((16,), jnp.float32)

        def zrow(i, carry):
            for jj in range(D // 16):
                rows_v[0, i, jj * 16:(jj + 1) * 16] = zero16
            return carry
        lax.fori_loop(0, CH, zrow, 0)
        for t in range(RPW // CH):
            pltpu.sync_copy(rows_v.at[0], acc.at[pl.ds(s * RPW + t * CH, CH)])
        pltpu.sync_copy(mask_hbm, mask_v)
        plsc.subcore_barrier()

        def fire_idx(g):
            bg = g % 2
            return [
                pltpu.async_copy(src_hbm.at[wid, pl.ds(g * 4, 4)],
                                 si_v.at[bg], sem_i),
                pltpu.async_copy(dst_hbm.at[wid, pl.ds(g * 4, 4)],
                                 di_v.at[bg], sem_i),
            ]

        iot = lax.iota(jnp.int32, 16)
        trash_src = lax.rem(iot * 523, 9973)
        trash_pk = (trash_src << 14) + (10001 + 2 * iot)
        BIG = jnp.int32(1 << 30)
        CB = CH * D * 4

        def unpack(off, half):
            for kk in range(8):
                pk = csrc_v[pl.ds(off + kk * 16, 16)]
                gidx_v[half, kk * 16:(kk + 1) * 16] = pk >> 14
                stage_v[half, kk * 16:(kk + 1) * 16] = pk & 16383

        def pair_body(i, carry):
            @pl.when(i >= 1)
            def _():
                pltpu.make_async_copy(h_hbm.at[pl.ds(0, CH)],
                                      rows_v.at[0], sem_sa).wait()
            unpack(i * 2 * CH, 0)
            ga = pltpu.async_copy(h_hbm.at[gidx_v.at[0]], rows_v.at[0],
                                  sem_ga)

            @pl.when(i >= 1)
            def _():
                pltpu.make_async_copy(h_hbm.at[pl.ds(0, CH)],
                                      rows_v.at[1], sem_sb).wait()
            unpack(i * 2 * CH + CH, 1)
            gb = pltpu.async_copy(h_hbm.at[gidx_v.at[1]], rows_v.at[1],
                                  sem_gb)
            ga.wait()
            pltpu.async_copy(rows_v.at[0], acc.at[stage_v.at[0]], sem_sa,
                             add=True)
            gb.wait()
            pltpu.async_copy(rows_v.at[1], acc.at[stage_v.at[1]], sem_sb,
                             add=True)
            return carry

        def drain_pairs(npairs):
            @pl.when(npairs >= 1)
            def _():
                pltpu.make_async_copy(h_hbm.at[pl.ds(0, CH)],
                                      rows_v.at[0], sem_sa).wait()
                pltpu.make_async_copy(h_hbm.at[pl.ds(0, CH)],
                                      rows_v.at[1], sem_sb).wait()

        def compact_group(bg, p):
            def cvec(v, pp):
                jj = v // 8
                kk = v % 8
                sv = si_v[bg, jj, pl.ds(kk * 16, 16)]
                dv = di_v[bg, jj, pl.ds(kk * 16, 16)]
                mv = plsc.load_gather(mask_v, [dv])
                m = mv != 0
                key = jnp.where(m, (sv << 14) + dv, BIG)
                sk, _ = plsc.sort_key_val(key, key)
                csrc_v[pl.ds(pp[0], 16)] = sk
                return pp + plsc.all_reduce_population_count(m)
            p = lax.fori_loop(0, 32, cvec, p)
            ps = p[0]
            npairs = ps // (2 * CH)
            lax.fori_loop(0, npairs, pair_body, 0)
            drain_pairs(npairs)
            for kk in range(16):
                vv = csrc_v[pl.ds(npairs * 2 * CH + kk * 16, 16)]
                csrc_v[pl.ds(kk * 16, 16)] = vv
            return jnp.zeros((16,), jnp.int32) + (ps - npairs * 2 * CH)

        idx_d = fire_idx(0)
        p = jnp.zeros((16,), jnp.int32)
        for g in range(NG2):
            for dd in idx_d:
                dd.wait()
            if g + 1 < NG2:
                idx_d = fire_idx(g + 1)
            p = compact_group(g % 2, p)
        ps = p[0]
        for kk in range(16):
            csrc_v[pl.ds(ps + kk * 16, 16)] = trash_pk
        nfp = (ps + 2 * CH - 1) // (2 * CH)
        lax.fori_loop(0, nfp, pair_body, 0)
        drain_pairs(nfp)

        plsc.subcore_barrier()
        pltpu.sync_copy(acc.at[pl.ds(s * RPW, RPW)],
                        agg_out.at[c, pl.ds(s * RPW, RPW)])

    return k(h, srcp, dstp, mask)[0]


def _sc_root(h2p, epack, ecnt):
    mesh = plsc.VectorSubcoreMesh(core_axis_name="c", subcore_axis_name="s")
    AROWS = 128
    arpw = AROWS // NS

    @functools.partial(
        pl.kernel, mesh=mesh,
        compiler_params=pltpu.CompilerParams(needs_layout_passes=False),
        out_type=(jax.ShapeDtypeStruct((NC, AROWS, D), jnp.float32),),
        scratch_types=[
            pltpu.VMEM((RING,), jnp.int32),
            pltpu.VMEM((CH,), jnp.int32),
            pltpu.VMEM((CH, D), jnp.float32),
            pltpu.VMEM((ZCH, D), jnp.float32),
            pltpu.VMEM((1, CH), jnp.int32),
            pltpu.VMEM((1, CH), jnp.int32),
            pltpu.SemaphoreType.DMA,
            pltpu.SemaphoreType.DMA,
            pltpu.VMEM_SHARED((AROWS, D), jnp.float32),
        ])
    def k(h_hbm, epack_hbm, ecnt_hbm, agg_out,
          pk_v, cnt_v, rows_v, zer_v, gidx_v, stage_v, sem_i, sem_g, acc):
        c = lax.axis_index("c")
        s = lax.axis_index("s")
        wid = c * NS + s

        zero16 = jnp.zeros((16,), jnp.float32)

        def zrow(i, carry):
            for jj in range(D // 16):
                zer_v[i, jj * 16:(jj + 1) * 16] = zero16
            return carry
        lax.fori_loop(0, arpw, zrow, 0)
        pltpu.sync_copy(zer_v.at[pl.ds(0, arpw)],
                        acc.at[pl.ds(s * arpw, arpw)])
        pltpu.sync_copy(ecnt_hbm.at[wid], cnt_v)
        plsc.subcore_barrier()

        nch = cnt_v[0:16][0]

        def proc_chunk(k2, carry):
            pltpu.async_copy(epack_hbm.at[wid, pl.ds(k2 * CH, CH)],
                             pk_v.at[pl.ds(0, CH)], sem_i).wait()
            for kk in range(8):
                pk = pk_v[pl.ds(kk * 16, 16)]
                gidx_v[0, kk * 16:(kk + 1) * 16] = pk >> 14
                stage_v[0, kk * 16:(kk + 1) * 16] = (pk & 16383) // 100
            pltpu.async_copy(h_hbm.at[gidx_v.at[0]], rows_v, sem_g).wait()
            pltpu.sync_copy(rows_v, acc.at[stage_v.at[0]], add=True)
            return carry
        lax.fori_loop(0, nch, proc_chunk, 0)

        plsc.subcore_barrier()
        pltpu.sync_copy(acc.at[pl.ds(s * arpw, arpw)],
                        agg_out.at[c, pl.ds(s * arpw, arpw)])

    return k(h2p, epack, ecnt)[0]


def _tc_stage(agg, cnt, h, Wl, Wr, b, maskp=None, pad_out=False):
    BR = 1000
    with_mask = maskp is not None

    def body(*refs):
        if with_mask:
            (a_ref, c_ref, h_ref, wl_ref, wr_ref, b_ref, m_ref,
             o_ref, mo_ref) = refs
        else:
            a_ref, c_ref, h_ref, wl_ref, wr_ref, b_ref, o_ref = refs
        a = a_ref[0] + a_ref[1]
        deg = c_ref[0] + c_ref[1]
        mean = a * (1.0 / jnp.maximum(deg, 1.0))
        o_ref[...] = (
            jnp.dot(mean, wl_ref[...], preferred_element_type=jnp.float32)
            + jnp.dot(h_ref[...], wr_ref[...], preferred_element_type=jnp.float32)
            + b_ref[...])
        if with_mask:
            node = (jax.lax.broadcasted_iota(jnp.int32, (BR, 1), 0)
                    + pl.program_id(0) * BR)
            needed = ((m_ref[0] + m_ref[1]) > 0.0) | (node % 100 == 0)
            mo_ref[...] = needed.astype(jnp.int32)

    in_specs = [
        pl.BlockSpec((NC, BR, D), lambda i: (0, i, 0)),
        pl.BlockSpec((NC, BR, 1), lambda i: (0, i, 0)),
        pl.BlockSpec((BR, D), lambda i: (i, 0)),
        pl.BlockSpec((D, D), lambda i: (0, 0)),
        pl.BlockSpec((D, D), lambda i: (0, 0)),
        pl.BlockSpec((1, D), lambda i: (0, 0)),
    ]
    out_shape = jax.ShapeDtypeStruct((NP_ if pad_out else N, D), jnp.float32)
    out_specs = pl.BlockSpec((BR, D), lambda i: (i, 0))
    args = [agg, cnt.reshape(NC, NP_, 1), h, Wl, Wr, b.reshape(1, D)]
    if with_mask:
        in_specs.append(pl.BlockSpec((NC, BR, 1), lambda i: (0, i, 0)))
        out_shape = [out_shape, jax.ShapeDtypeStruct((N, 1), jnp.int32)]
        out_specs = [out_specs, pl.BlockSpec((BR, 1), lambda i: (i, 0))]
        args.append(maskp.reshape(NC, NP_, 1))

    return pl.pallas_call(
        body,
        grid=(N // BR,),
        in_specs=in_specs,
        out_specs=out_specs,
        out_shape=out_shape,
    )(*args)


def _tc_head(aggr, cntr, h2r, Wl2, Wr2, b2, Wm1, bm1, Wm2, bm2):
    def body(a_ref, c_ref, h_ref, wl_ref, wr_ref, b_ref,
             w1_ref, b1_ref, w2_ref, b2_ref, o_ref):
        a = a_ref[0, 0:B, :] + a_ref[1, 0:B, :]
        deg = c_ref[0] + c_ref[1]
        mean = a * (1.0 / jnp.maximum(deg, 1.0))
        h3 = (jnp.dot(mean, wl_ref[...], preferred_element_type=jnp.float32)
              + jnp.dot(h_ref[...], wr_ref[...], preferred_element_type=jnp.float32)
              + b_ref[...])
        z = jnp.maximum(
            jnp.dot(h3, w1_ref[...], preferred_element_type=jnp.float32)
            + b1_ref[...], 0.0)
        o_ref[...] = (jnp.dot(z, w2_ref[...], preferred_element_type=jnp.float32)
                      + b2_ref[...])

    return pl.pallas_call(
        body,
        out_shape=jax.ShapeDtypeStruct((B, OUT), jnp.float32),
    )(aggr, cntr, h2r, Wl2, Wr2, b2.reshape(1, D),
      Wm1, bm1.reshape(1, MLP_H), Wm2, bm2.reshape(1, OUT))


def kernel(x, edge_index, Wl0, Wr0, b0, Wl1, Wr1, b1, Wl2, Wr2, b2,
           Wm1, bm1, Wm2, bm2):
    ei = edge_index.astype(jnp.int32)
    npad = E_PAD - E
    ar = jnp.arange(npad, dtype=jnp.int32)
    pad_src = lax.rem(ar * 13, N)
    pad_dst = 10001 + 2 * lax.rem(ar, 119)
    srcp = jnp.concatenate([ei[0], pad_src]).reshape(NW, NCHUNK, CH)
    dstp = jnp.concatenate([ei[1], pad_dst]).reshape(NW, NCHUNK, CH)

    agg1, cnt, maskp, epack, ecnt = _sc_l1(x, srcp, dstp)
    h1, mask = _tc_stage(agg1, cnt, x, Wl0, Wr0, b0, maskp=maskp)
    maskf = jnp.concatenate([mask.reshape(N),
                             jnp.zeros((NP_ - N,), jnp.int32)])
    agg2 = _sc_pruned(h1, srcp, dstp, maskf)
    h2p = _tc_stage(agg2, cnt, h1, Wl1, Wr1, b1, pad_out=True)
    agg3r = _sc_root(h2p, epack, ecnt)

    h2r = h2p[0:N:B]
    cntr = cnt[:, :N].reshape(NC, B, N // B)[:, :, 0].reshape(NC, B, 1)
    return _tc_head(agg3r, cntr, h2r, Wl2, Wr2, b2, Wm1, bm1, Wm2, bm2)


# speedup vs baseline: 16.4670x; 1.0117x over previous
import functools

import jax
import jax.numpy as jnp
from jax import lax
from jax.experimental import pallas as pl
from jax.experimental.pallas import tpu as pltpu
from jax.experimental.pallas import tpu_sc as plsc

N = 10000
E = 320000
D = 128
B = 100
MLP_H = 256
OUT = 64

NC = 2
NS = 16
NW = NC * NS
CH = 128
NCHUNK = 80
EPT = NCHUNK * CH
E_PAD = NW * EPT
NG = NCHUNK // 8
NP_ = 10240
RPW = NP_ // NS
ZCH = 32
RING = 1280
EPC = EPT + CH


def _sc_l1(h, srcp, dstp):
    mesh = plsc.VectorSubcoreMesh(core_axis_name="c", subcore_axis_name="s")

    @functools.partial(
        pl.kernel, mesh=mesh,
        compiler_params=pltpu.CompilerParams(needs_layout_passes=False),
        out_type=(jax.ShapeDtypeStruct((NC, NP_, D), jnp.float32),
                  jax.ShapeDtypeStruct((NC, NP_), jnp.float32),
                  jax.ShapeDtypeStruct((NC, NP_), jnp.float32),
                  jax.ShapeDtypeStruct((NW, EPC), jnp.int32),
                  jax.ShapeDtypeStruct((NW, CH), jnp.int32)),
        scratch_types=[
            pltpu.VMEM((2, 8, CH), jnp.int32),
            pltpu.VMEM((2, 8, CH), jnp.int32),
            pltpu.VMEM((2, CH, D), jnp.float32),
            pltpu.VMEM((ZCH, D), jnp.float32),
            pltpu.VMEM((RING,), jnp.int32),
            pltpu.VMEM((1, CH), jnp.int32),
            pltpu.VMEM((CH,), jnp.float32),
            pltpu.VMEM((RPW,), jnp.float32),
            pltpu.SemaphoreType.DMA,
            pltpu.SemaphoreType.DMA,
            pltpu.SemaphoreType.DMA,
            pltpu.SemaphoreType.DMA,
            pltpu.SemaphoreType.DMA,
            pltpu.SemaphoreType.DMA,
            pltpu.SemaphoreType.DMA,
            pltpu.VMEM_SHARED((NP_, D), jnp.float32),
            pltpu.VMEM_SHARED((NP_,), jnp.float32),
            pltpu.VMEM_SHARED((NP_,), jnp.float32),
        ])
    def k(h_hbm, src_hbm, dst_hbm, agg_out, cnt_out, mask_out,
          epack_out, ecnt_out,
          si_v, di_v, rows_v, zer_v, csrc_v, stage_v, ones_v, z16_v,
          sem_i, sg0, sg1, ss0, ss1, sc0, sc1, acc, cacc, macc):
        sem_g = (sg0, sg1)
        sem_s = (ss0, ss1)
        sem_c = (sc0, sc1)
        c = lax.axis_index("c")
        s = lax.axis_index("s")
        wid = c * NS + s

        zero16 = jnp.zeros((16,), jnp.float32)
        one16 = jnp.ones((16,), jnp.float32)

        def zrow(i, carry):
            for jj in range(D // 16):
                zer_v[i, jj * 16:(jj + 1) * 16] = zero16
            return carry
        lax.fori_loop(0, ZCH, zrow, 0)
        for t in range(RPW // ZCH):
            pltpu.sync_copy(zer_v, acc.at[pl.ds(s * RPW + t * ZCH, ZCH)])
        for jj in range(RPW // 16):
            z16_v[jj * 16:(jj + 1) * 16] = zero16
        for jj in range(CH // 16):
            ones_v[jj * 16:(jj + 1) * 16] = one16
        pltpu.sync_copy(z16_v, cacc.at[pl.ds(s * RPW, RPW)])
        pltpu.sync_copy(z16_v, macc.at[pl.ds(s * RPW, RPW)])
        plsc.subcore_barrier()

        gat = [None, None]
        sca = [None, None]

        def fire_idx(g):
            bg = g % 2
            return [
                pltpu.async_copy(src_hbm.at[wid, pl.ds(g * 8, 8)],
                                 si_v.at[bg], sem_i),
                pltpu.async_copy(dst_hbm.at[wid, pl.ds(g * 8, 8)],
                                 di_v.at[bg], sem_i),
            ]

        idx_d = fire_idx(0)

        def start_gather(j):
            g, jj = divmod(j, 8)
            br = j % 2
            gat[br] = pltpu.async_copy(
                h_hbm.at[si_v.at[g % 2, jj]], rows_v.at[br], sem_g[br])

        def start_scatter(j):
            g, jj = divmod(j, 8)
            br = j % 2
            sca[br] = [
                pltpu.async_copy(rows_v.at[br], acc.at[di_v.at[g % 2, jj]],
                                 sem_s[br], add=True),
                pltpu.async_copy(ones_v, cacc.at[di_v.at[g % 2, jj]],
                                 sem_c[br], add=True),
            ]

        iot = lax.iota(jnp.int32, 16)
        trash_pk = ((10001 + 2 * iot) << 14) + (10001 + 2 * iot)
        mBIG = jnp.int32(1 << 30)

        def mask_chunk(k2, gk):
            for kk in range(8):
                vv = csrc_v[pl.ds(k2 * CH + kk * 16, 16)]
                stage_v[0, kk * 16:(kk + 1) * 16] = vv >> 14
            pltpu.sync_copy(ones_v, macc.at[stage_v.at[0]], add=True)
            pltpu.sync_copy(csrc_v.at[pl.ds(k2 * CH, CH)],
                            epack_out.at[wid, pl.ds(gk * CH, CH)])
            return gk + 1

        def mask_compact(bg, p, gk):
            def cvec(v, pp):
                jj = v // 8
                kk = v % 8
                sv = si_v[bg, jj, pl.ds(kk * 16, 16)]
                dv = di_v[bg, jj, pl.ds(kk * 16, 16)]
                m = lax.rem(dv, 100) == 0
                key = jnp.where(m, (sv << 14) + dv, mBIG)
                sk, _ = plsc.sort_key_val(key, key)
                csrc_v[pl.ds(pp[0], 16)] = sk
                return pp + plsc.all_reduce_population_count(m)
            p = lax.fori_loop(0, 64, cvec, p)
            ps = p[0]
            nfull = ps // CH
            gk = lax.fori_loop(0, nfull, mask_chunk, gk)
            for kk in range(8):
                vv = csrc_v[pl.ds(nfull * CH + kk * 16, 16)]
                csrc_v[pl.ds(kk * 16, 16)] = vv
            return jnp.zeros((16,), jnp.int32) + (ps - nfull * CH), gk

        def mask_finish(p, gk):
            ps = p[0]
            for kk in range(8):
                csrc_v[pl.ds(ps + kk * 16, 16)] = trash_pk
            gk = lax.fori_loop(0, (ps + CH - 1) // CH, mask_chunk, gk)
            for kk in range(8):
                stage_v[0, kk * 16:(kk + 1) * 16] = iot * 0 + gk
            pltpu.sync_copy(stage_v.at[0], ecnt_out.at[wid])

        mp = jnp.zeros((16,), jnp.int32)
        mgk = jnp.int32(0)
        for g in range(NG):
            for jj in range(8):
                j = g * 8 + jj
                br = j % 2
                if sca[br] is not None:
                    for dd in sca[br]:
                        dd.wait()
                    sca[br] = None
                if jj == 0:
                    for dd in idx_d:
                        dd.wait()
                    idx_d = []
                    mp, mgk = mask_compact(g % 2, mp, mgk)
                start_gather(j)
                if jj == 1 and g + 1 < NG:
                    idx_d = fire_idx(g + 1)
                if j >= 1:
                    gat[1 - br].wait()
                    start_scatter(j - 1)
        last = NCHUNK - 1
        gat[last % 2].wait()
        start_scatter(last)
        mask_finish(mp, mgk)
        for bb in range(2):
            if sca[bb] is not None:
                for dd in sca[bb]:
                    dd.wait()

        plsc.subcore_barrier()
        pltpu.sync_copy(acc.at[pl.ds(s * RPW, RPW)],
                        agg_out.at[c, pl.ds(s * RPW, RPW)])
        pltpu.sync_copy(cacc.at[pl.ds(s * RPW, RPW)],
                        cnt_out.at[c, pl.ds(s * RPW, RPW)])
        pltpu.sync_copy(macc.at[pl.ds(s * RPW, RPW)],
                        mask_out.at[c, pl.ds(s * RPW, RPW)])

    return k(h, srcp, dstp)


def _sc_pruned(h, srcp, dstp, mask):
    mesh = plsc.VectorSubcoreMesh(core_axis_name="c", subcore_axis_name="s")
    NG2 = NCHUNK // 4

    @functools.partial(
        pl.kernel, mesh=mesh,
        compiler_params=pltpu.CompilerParams(needs_layout_passes=False),
        out_type=(jax.ShapeDtypeStruct((NC, NP_, D), jnp.float32),),
        scratch_types=[
            pltpu.VMEM((2, 4, CH), jnp.int32),
            pltpu.VMEM((2, 4, CH), jnp.int32),
            pltpu.VMEM((2, CH, D), jnp.float32),
            pltpu.VMEM((RING,), jnp.int32),
            pltpu.VMEM((2, CH), jnp.int32),
            pltpu.VMEM((2, CH), jnp.int32),
            pltpu.SemaphoreType.DMA,
            pltpu.SemaphoreType.DMA,
            pltpu.SemaphoreType.DMA,
            pltpu.SemaphoreType.DMA,
            pltpu.SemaphoreType.DMA,
            pltpu.VMEM_SHARED((NP_, D), jnp.float32),
            pltpu.VMEM((NP_,), jnp.int32),
        ])
    def k(h_hbm, src_hbm, dst_hbm, mask_hbm, agg_out,
          si_v, di_v, rows_v, csrc_v, gidx_v, stage_v,
          sem_i, sem_ga, sem_gb, sem_sa, sem_sb, acc, mask_v):
        c = lax.axis_index("c")
        s = lax.axis_index("s")
        wid = c * NS + s

        zero16 = jnp.zeros((16,), jnp.float32)

        def zrow(i, carry):
            for jj in range(D // 16):
                rows_v[0, i, jj * 16:(jj + 1) * 16] = zero16
            return carry
        lax.fori_loop(0, CH, zrow, 0)
        for t in range(RPW // CH):
            pltpu.sync_copy(rows_v.at[0], acc.at[pl.ds(s * RPW + t * CH, CH)])
        pltpu.sync_copy(mask_hbm, mask_v)
        plsc.subcore_barrier()

        def fire_idx(g):
            bg = g % 2
            return [
                pltpu.async_copy(src_hbm.at[wid, pl.ds(g * 4, 4)],
                                 si_v.at[bg], sem_i),
                pltpu.async_copy(dst_hbm.at[wid, pl.ds(g * 4, 4)],
                                 di_v.at[bg], sem_i),
            ]

        iot = lax.iota(jnp.int32, 16)
        trash_src = lax.rem(iot * 523, 9973)
        trash_pk = (trash_src << 14) + (10001 + 2 * iot)
        BIG = jnp.int32(1 << 30)
        CB = CH * D * 4

        def unpack(off, half):
            for kk in range(8):
                pk = csrc_v[pl.ds(off + kk * 16, 16)]
                gidx_v[half, kk * 16:(kk + 1) * 16] = pk >> 14
                stage_v[half, kk * 16:(kk + 1) * 16] = pk & 16383

        def pair_body(i, carry):
            @pl.when(i >= 1)
            def _():
                pltpu.make_async_copy(h_hbm.at[pl.ds(0, CH)],
                                      rows_v.at[0], sem_sa).wait()
            unpack(i * 2 * CH, 0)
            ga = pltpu.async_copy(h_hbm.at[gidx_v.at[0]], rows_v.at[0],
                                  sem_ga)

            @pl.when(i >= 1)
            def _():
                pltpu.make_async_copy(h_hbm.at[pl.ds(0, CH)],
                                      rows_v.at[1], sem_sb).wait()
            unpack(i * 2 * CH + CH, 1)
            gb = pltpu.async_copy(h_hbm.at[gidx_v.at[1]], rows_v.at[1],
                                  sem_gb)
            ga.wait()
            pltpu.async_copy(rows_v.at[0], acc.at[stage_v.at[0]], sem_sa,
                             add=True)
            gb.wait()
            pltpu.async_copy(rows_v.at[1], acc.at[stage_v.at[1]], sem_sb,
                             add=True)
            return carry

        def drain_pairs(npairs):
            @pl.when(npairs >= 1)
            def _():
                pltpu.make_async_copy(h_hbm.at[pl.ds(0, CH)],
                                      rows_v.at[0], sem_sa).wait()
                pltpu.make_async_copy(h_hbm.at[pl.ds(0, CH)],
                                      rows_v.at[1], sem_sb).wait()

        def compact_group(bg, p):
            def cvec2(v2, pp):
                jj = v2 // 4
                kk = (v2 % 4) * 2
                sva = si_v[bg, jj, pl.ds(kk * 16, 16)]
                dva = di_v[bg, jj, pl.ds(kk * 16, 16)]
                svb = si_v[bg, jj, pl.ds(kk * 16 + 16, 16)]
                dvb = di_v[bg, jj, pl.ds(kk * 16 + 16, 16)]
                mva = plsc.load_gather(mask_v, [dva])
                mvb = plsc.load_gather(mask_v, [dvb])
                ma = mva != 0
                mb = mvb != 0
                keya = jnp.where(ma, (sva << 14) + dva, BIG)
                keyb = jnp.where(mb, (svb << 14) + dvb, BIG)
                ska, _ = plsc.sort_key_val(keya, keya)
                skb, _ = plsc.sort_key_val(keyb, keyb)
                pa = pp[0]
                ta = plsc.all_reduce_population_count(ma)
                csrc_v[pl.ds(pa, 16)] = ska
                pb = (pp + ta)[0]
                csrc_v[pl.ds(pb, 16)] = skb
                return pp + ta + plsc.all_reduce_population_count(mb)
            p = lax.fori_loop(0, 16, cvec2, p)
            ps = p[0]
            npairs = ps // (2 * CH)
            lax.fori_loop(0, npairs, pair_body, 0)
            drain_pairs(npairs)
            for kk in range(16):
                vv = csrc_v[pl.ds(npairs * 2 * CH + kk * 16, 16)]
                csrc_v[pl.ds(kk * 16, 16)] = vv
            return jnp.zeros((16,), jnp.int32) + (ps - npairs * 2 * CH)

        idx_d = fire_idx(0)
        p = jnp.zeros((16,), jnp.int32)
        for g in range(NG2):
            for dd in idx_d:
                dd.wait()
            if g + 1 < NG2:
                idx_d = fire_idx(g + 1)
            p = compact_group(g % 2, p)
        ps = p[0]
        for kk in range(16):
            csrc_v[pl.ds(ps + kk * 16, 16)] = trash_pk
        nfp = (ps + 2 * CH - 1) // (2 * CH)
        lax.fori_loop(0, nfp, pair_body, 0)
        drain_pairs(nfp)

        plsc.subcore_barrier()
        pltpu.sync_copy(acc.at[pl.ds(s * RPW, RPW)],
                        agg_out.at[c, pl.ds(s * RPW, RPW)])

    return k(h, srcp, dstp, mask)[0]


def _sc_root(h2p, epack, ecnt):
    mesh = plsc.VectorSubcoreMesh(core_axis_name="c", subcore_axis_name="s")
    AROWS = 128
    arpw = AROWS // NS

    @functools.partial(
        pl.kernel, mesh=mesh,
        compiler_params=pltpu.CompilerParams(needs_layout_passes=False),
        out_type=(jax.ShapeDtypeStruct((NC, AROWS, D), jnp.float32),),
        scratch_types=[
            pltpu.VMEM((RING,), jnp.int32),
            pltpu.VMEM((CH,), jnp.int32),
            pltpu.VMEM((CH, D), jnp.float32),
            pltpu.VMEM((ZCH, D), jnp.float32),
            pltpu.VMEM((1, CH), jnp.int32),
            pltpu.VMEM((1, CH), jnp.int32),
            pltpu.SemaphoreType.DMA,
            pltpu.SemaphoreType.DMA,
            pltpu.VMEM_SHARED((AROWS, D), jnp.float32),
        ])
    def k(h_hbm, epack_hbm, ecnt_hbm, agg_out,
          pk_v, cnt_v, rows_v, zer_v, gidx_v, stage_v, sem_i, sem_g, acc):
        c = lax.axis_index("c")
        s = lax.axis_index("s")
        wid = c * NS + s

        zero16 = jnp.zeros((16,), jnp.float32)

        def zrow(i, carry):
            for jj in range(D // 16):
                zer_v[i, jj * 16:(jj + 1) * 16] = zero16
            return carry
        lax.fori_loop(0, arpw, zrow, 0)
        pltpu.sync_copy(zer_v.at[pl.ds(0, arpw)],
                        acc.at[pl.ds(s * arpw, arpw)])
        pltpu.sync_copy(ecnt_hbm.at[wid], cnt_v)
        plsc.subcore_barrier()

        nch = cnt_v[0:16][0]

        def proc_chunk(k2, carry):
            pltpu.async_copy(epack_hbm.at[wid, pl.ds(k2 * CH, CH)],
                             pk_v.at[pl.ds(0, CH)], sem_i).wait()
            for kk in range(8):
                pk = pk_v[pl.ds(kk * 16, 16)]
                gidx_v[0, kk * 16:(kk + 1) * 16] = pk >> 14
                stage_v[0, kk * 16:(kk + 1) * 16] = (pk & 16383) // 100
            pltpu.async_copy(h_hbm.at[gidx_v.at[0]], rows_v, sem_g).wait()
            pltpu.sync_copy(rows_v, acc.at[stage_v.at[0]], add=True)
            return carry
        lax.fori_loop(0, nch, proc_chunk, 0)

        plsc.subcore_barrier()
        pltpu.sync_copy(acc.at[pl.ds(s * arpw, arpw)],
                        agg_out.at[c, pl.ds(s * arpw, arpw)])

    return k(h2p, epack, ecnt)[0]


def _tc_stage(agg, cnt, h, Wl, Wr, b, maskp=None, pad_out=False):
    BR = 1000
    with_mask = maskp is not None

    def body(*refs):
        if with_mask:
            (a_ref, c_ref, h_ref, wl_ref, wr_ref, b_ref, m_ref,
             o_ref, mo_ref) = refs
        else:
            a_ref, c_ref, h_ref, wl_ref, wr_ref, b_ref, o_ref = refs
        a = a_ref[0] + a_ref[1]
        deg = c_ref[0] + c_ref[1]
        mean = a * (1.0 / jnp.maximum(deg, 1.0))
        o_ref[...] = (
            jnp.dot(mean, wl_ref[...], preferred_element_type=jnp.float32)
            + jnp.dot(h_ref[...], wr_ref[...], preferred_element_type=jnp.float32)
            + b_ref[...])
        if with_mask:
            node = (jax.lax.broadcasted_iota(jnp.int32, (BR, 1), 0)
                    + pl.program_id(0) * BR)
            needed = ((m_ref[0] + m_ref[1]) > 0.0) | (node % 100 == 0)
            mo_ref[...] = needed.astype(jnp.int32)

    in_specs = [
        pl.BlockSpec((NC, BR, D), lambda i: (0, i, 0)),
        pl.BlockSpec((NC, BR, 1), lambda i: (0, i, 0)),
        pl.BlockSpec((BR, D), lambda i: (i, 0)),
        pl.BlockSpec((D, D), lambda i: (0, 0)),
        pl.BlockSpec((D, D), lambda i: (0, 0)),
        pl.BlockSpec((1, D), lambda i: (0, 0)),
    ]
    out_shape = jax.ShapeDtypeStruct((NP_ if pad_out else N, D), jnp.float32)
    out_specs = pl.BlockSpec((BR, D), lambda i: (i, 0))
    args = [agg, cnt.reshape(NC, NP_, 1), h, Wl, Wr, b.reshape(1, D)]
    if with_mask:
        in_specs.append(pl.BlockSpec((NC, BR, 1), lambda i: (0, i, 0)))
        out_shape = [out_shape, jax.ShapeDtypeStruct((N, 1), jnp.int32)]
        out_specs = [out_specs, pl.BlockSpec((BR, 1), lambda i: (i, 0))]
        args.append(maskp.reshape(NC, NP_, 1))

    return pl.pallas_call(
        body,
        grid=(N // BR,),
        in_specs=in_specs,
        out_specs=out_specs,
        out_shape=out_shape,
    )(*args)


def _tc_head(aggr, cntr, h2r, Wl2, Wr2, b2, Wm1, bm1, Wm2, bm2):
    def body(a_ref, c_ref, h_ref, wl_ref, wr_ref, b_ref,
             w1_ref, b1_ref, w2_ref, b2_ref, o_ref):
        a = a_ref[0, 0:B, :] + a_ref[1, 0:B, :]
        deg = c_ref[0] + c_ref[1]
        mean = a * (1.0 / jnp.maximum(deg, 1.0))
        h3 = (jnp.dot(mean, wl_ref[...], preferred_element_type=jnp.float32)
              + jnp.dot(h_ref[...], wr_ref[...], preferred_element_type=jnp.float32)
              + b_ref[...])
        z = jnp.maximum(
            jnp.dot(h3, w1_ref[...], preferred_element_type=jnp.float32)
            + b1_ref[...], 0.0)
        o_ref[...] = (jnp.dot(z, w2_ref[...], preferred_element_type=jnp.float32)
                      + b2_ref[...])

    return pl.pallas_call(
        body,
        out_shape=jax.ShapeDtypeStruct((B, OUT), jnp.float32),
    )(aggr, cntr, h2r, Wl2, Wr2, b2.reshape(1, D),
      Wm1, bm1.reshape(1, MLP_H), Wm2, bm2.reshape(1, OUT))


def kernel(x, edge_index, Wl0, Wr0, b0, Wl1, Wr1, b1, Wl2, Wr2, b2,
           Wm1, bm1, Wm2, bm2):
    ei = edge_index.astype(jnp.int32)
    npad = E_PAD - E
    ar = jnp.arange(npad, dtype=jnp.int32)
    pad_src = lax.rem(ar * 13, N)
    pad_dst = 10001 + 2 * lax.rem(ar, 119)
    srcp = jnp.concatenate([ei[0], pad_src]).reshape(NW, NCHUNK, CH)
    dstp = jnp.concatenate([ei[1], pad_dst]).reshape(NW, NCHUNK, CH)

    agg1, cnt, maskp, epack, ecnt = _sc_l1(x, srcp, dstp)
    h1, mask = _tc_stage(agg1, cnt, x, Wl0, Wr0, b0, maskp=maskp)
    maskf = jnp.concatenate([mask.reshape(N),
                             jnp.zeros((NP_ - N,), jnp.int32)])
    agg2 = _sc_pruned(h1, srcp, dstp, maskf)
    h2p = _tc_stage(agg2, cnt, h1, Wl1, Wr1, b1, pad_out=True)
    agg3r = _sc_root(h2p, epack, ecnt)

    h2r = h2p[0:N:B]
    cntr = cnt[:, :N].reshape(NC, B, N // B)[:, :, 0].reshape(NC, B, 1)
    return _tc_head(agg3r, cntr, h2r, Wl2, Wr2, b2, Wm1, bm1, Wm2, bm2)
